# baseline (speedup 1.0000x reference)
import functools

import jax
import jax.numpy as jnp
from jax import lax
from jax.experimental import pallas as pl
from jax.experimental.pallas import tpu as pltpu
from jax.experimental.pallas import tpu_sc as plsc

N = 10000
E = 320000
NF = 128
NH = 128
HEADS = 2
R = 16
ALPHA = 0.2
K = 3

NP = 10240
NC = 2
NS = 16
L = 16
C = 128
EPT = 20480
EPAD = NS * EPT
NCHUNK = EPT // C
RPT = NP // NS
RCH = 64
NRC = RPT // RCH
AC = NH + L


_BR = 1024


def _dot(a, b, dims):
    return lax.dot_general(a, b, (dims, ((), ())),
                           preferred_element_type=jnp.float32)


def _prep_body(x_ref, rel_ref, w1_ref, b1_ref, w2_ref, b2_ref, cr_ref,
               wa_ref, wr_ref, aa_ref, wo_ref,
               t_ref, s3_ref, xw_ref, xrel_ref):
    xb = x_ref[...]
    rel = rel_ref[...]
    for h in range(HEADS):
        hid = jnp.maximum(_dot(rel, w1_ref[h], ((1,), (0,))) + b1_ref[h], 0.0)
        theta = _dot(hid, w2_ref[h], ((1,), (0,))) + b2_ref[h]
        ct = jnp.cos(theta)
        a = aa_ref[h]
        a1 = a[:, 0:NH]
        a2 = a[:, NH:2 * NH]
        a3 = a[:, 2 * NH:3 * NH]
        v1 = _dot(a1, wa_ref[h], ((1,), (1,)))
        v2 = _dot(a2, wr_ref[h], ((1,), (1,)))
        v3 = _dot(a3, wa_ref[h], ((1,), (1,)))
        s1 = _dot(xb, v1, ((1,), (1,)))
        s2 = _dot(xb, ct * v2, ((1,), (1,)))
        cterm = _dot(cr_ref[h], v2, ((1,), (1,)))
        t_ref[h] = s1 + s2 + cterm
        s3_ref[h] = _dot(xb, v3, ((1,), (1,)))
        xw_ref[h] = _dot(xb, wa_ref[h], ((1,), (0,)))
    xrel_ref[...] = _dot(rel, wo_ref[...], ((1,), (0,)))


def _prep(x_pad, rel, w1, b1, w2, b2, cr, wa, wr, aa, wo):
    full = lambda shape: pl.BlockSpec(shape, lambda i: tuple(0 for _ in shape))
    grid = NP // _BR
    return pl.pallas_call(
        _prep_body,
        grid=(grid,),
        in_specs=[
            pl.BlockSpec((_BR, NF), lambda i: (i, 0)),
            full((R, NF)),
            full((HEADS, NF, 512)),
            full((HEADS, 1, 512)),
            full((HEADS, 512, NF)),
            full((HEADS, 1, NF)),
            full((HEADS, 1, NF)),
            full((HEADS, NF, NH)),
            full((HEADS, NF, NH)),
            full((HEADS, 1, 3 * NH)),
            full((NF, HEADS * NH)),
        ],
        out_specs=[
            pl.BlockSpec((HEADS, _BR, R), lambda i: (0, i, 0)),
            pl.BlockSpec((HEADS, _BR, 1), lambda i: (0, i, 0)),
            pl.BlockSpec((HEADS, _BR, NH), lambda i: (0, i, 0)),
            pl.BlockSpec((R, HEADS * NH), lambda i: (0, 0)),
        ],
        out_shape=[
            jax.ShapeDtypeStruct((HEADS, NP, R), jnp.float32),
            jax.ShapeDtypeStruct((HEADS, NP, 1), jnp.float32),
            jax.ShapeDtypeStruct((HEADS, NP, NH), jnp.float32),
            jax.ShapeDtypeStruct((R, HEADS * NH), jnp.float32),
        ],
    )(x_pad, rel, w1, b1, w2, b2, cr, wa, wr, aa, wo)


_MESH = plsc.VectorSubcoreMesh(core_axis_name="c", subcore_axis_name="s")


def _zero_buf(buf, rows, cols):
    def zrow(i, carry):
        for j in range(cols // L):
            buf[i, pl.ds(j * L, L)] = jnp.zeros((L,), jnp.float32)
        return carry
    lax.fori_loop(0, rows, zrow, 0)


def _k1_body(src_h, dst_h, et_h, t_h, s3_h, xw_h,
             hl_h, hls_h, dis_h,
             acc_sh, rsum_sh, deg_sh,
             s3_v, isrc, idst, iet, iflat, ixw, tval, xrow, e_v, ones_v,
             fin, disc, invc, rs_v, deg_v, sem):
    c = lax.axis_index("c")
    s = lax.axis_index("s")
    lane = lax.iota(jnp.int32, L)

    _zero_buf(fin, RCH, NH)
    for gg in range(RCH // L):
        rs_v[pl.ds(gg * L, L)] = jnp.zeros((L,), jnp.float32)
    for grp in range(C // L):
        ones_v[pl.ds(grp * L, L)] = jnp.ones((L,), jnp.float32)
    for k in range(NRC):
        rb = s * RPT + k * RCH
        pltpu.sync_copy(fin, acc_sh.at[pl.ds(rb, RCH)])
        pltpu.sync_copy(rs_v, rsum_sh.at[pl.ds(rb, RCH)])
        pltpu.sync_copy(rs_v, deg_sh.at[pl.ds(rb, RCH)])
    pltpu.sync_copy(s3_h.at[pl.ds(c * NP, NP)], s3_v)
    plsc.subcore_barrier()

    def edge_chunk(g, carry):
        base = s * EPT + g * C
        pltpu.sync_copy(src_h.at[pl.ds(base, C)], isrc)
        pltpu.sync_copy(dst_h.at[pl.ds(base, C)], idst)
        pltpu.sync_copy(et_h.at[pl.ds(base, C)], iet)
        for grp in range(C // L):
            sg = isrc[pl.ds(grp * L, L)]
            etg = iet[pl.ds(grp * L, L)]
            dg = idst[pl.ds(grp * L, L)]
            iflat[pl.ds(grp * L, L)] = (sg + c * NP) * R + etg
            ixw[pl.ds(grp * L, L)] = dg + c * NP
        pltpu.async_copy(t_h.at[iflat], tval, sem).wait()
        pltpu.async_copy(xw_h.at[ixw], xrow, sem).wait()
        for grp in range(C // L):
            dstg = idst[pl.ds(grp * L, L)]
            tv = tval[pl.ds(grp * L, L)]
            s3d = plsc.load_gather(s3_v, [dstg])
            logit = tv + s3d
            lr = jnp.where(logit >= 0.0, logit, ALPHA * logit)
            e_v[pl.ds(grp * L, L)] = jnp.exp(-lr)

        def scale_row(i, carry2):
            ei = e_v[pl.ds(i, L)][0]
            for j in range(NH // L):
                xrow[i, pl.ds(j * L, L)] = xrow[i, pl.ds(j * L, L)] * ei
            return carry2
        lax.fori_loop(0, C, scale_row, 0)
        pltpu.sync_copy(xrow, acc_sh.at[isrc], add=True)
        pltpu.sync_copy(e_v.at[pl.ds(0, C)], rsum_sh.at[isrc], add=True)
        pltpu.sync_copy(ones_v, deg_sh.at[isrc], add=True)
        return carry
    lax.fori_loop(0, NCHUNK, edge_chunk, 0)
    plsc.subcore_barrier()

    for k in range(NRC):
        rb = s * RPT + k * RCH
        pltpu.sync_copy(acc_sh.at[pl.ds(rb, RCH)], fin)
        pltpu.sync_copy(rsum_sh.at[pl.ds(rb, RCH)], rs_v)
        pltpu.sync_copy(deg_sh.at[pl.ds(rb, RCH)], deg_v)
        for gg in range(RCH // L):
            rs = rs_v[pl.ds(gg * L, L)]
            rs = jnp.where(rs == 0.0, 1e-12, rs)
            degv = deg_v[pl.ds(gg * L, L)] + 1.0
            ii = plsc.bitcast(degv, jnp.int32)
            ii = jnp.int32(0x5F3759DF) - lax.shift_right_arithmetic(ii, 1)
            y = plsc.bitcast(ii, jnp.float32)
            for _ in range(3):
                y = y * (1.5 - 0.5 * degv * y * y)
            invc[pl.ds(gg * L, L)] = 1.0 / rs
            disc[pl.ds(gg * L, L)] = y

        def fin_row(i, carry):
            iv = invc[pl.ds(i, L)][0]
            for j in range(NH // L):
                v = fin[i, pl.ds(j * L, L)] * iv
                fin[i, pl.ds(j * L, L)] = jnp.where(v > 0.0, v,
                                                    jnp.exp(v) - 1.0)
            return carry
        lax.fori_loop(0, RCH, fin_row, 0)
        pltpu.sync_copy(fin, hl_h.at[pl.ds(c * NP + rb, RCH)])

        def scl_row(i, carry):
            di = disc[pl.ds(i, L)][0]
            for j in range(NH // L):
                fin[i, pl.ds(j * L, L)] = fin[i, pl.ds(j * L, L)] * di
            return carry
        lax.fori_loop(0, RCH, scl_row, 0)
        pltpu.sync_copy(fin, hls_h.at[pl.ds(c * NP + rb, RCH)])

        @pl.when(c == 0)
        def _():
            pltpu.sync_copy(disc.at[pl.ds(0, RCH)], dis_h.at[pl.ds(rb, RCH)])


_k1 = functools.partial(
    pl.kernel,
    out_type=(jax.ShapeDtypeStruct((HEADS * NP, NH), jnp.float32),
              jax.ShapeDtypeStruct((HEADS * NP, NH), jnp.float32),
              jax.ShapeDtypeStruct((NP,), jnp.float32)),
    mesh=_MESH,
    compiler_params=pltpu.CompilerParams(needs_layout_passes=False),
    scratch_types=[
        pltpu.VMEM_SHARED((NP, NH), jnp.float32),
        pltpu.VMEM_SHARED((NP,), jnp.float32),
        pltpu.VMEM_SHARED((NP,), jnp.float32),
        pltpu.VMEM((NP,), jnp.float32),
        pltpu.VMEM((C,), jnp.int32),
        pltpu.VMEM((C,), jnp.int32),
        pltpu.VMEM((C,), jnp.int32),
        pltpu.VMEM((C,), jnp.int32),
        pltpu.VMEM((C,), jnp.int32),
        pltpu.VMEM((C,), jnp.float32),
        pltpu.VMEM((C, NH), jnp.float32),
        pltpu.VMEM((C + L,), jnp.float32),
        pltpu.VMEM((C,), jnp.float32),
        pltpu.VMEM((RCH, NH), jnp.float32),
        pltpu.VMEM((RCH + L,), jnp.float32),
        pltpu.VMEM((RCH + L,), jnp.float32),
        pltpu.VMEM((RCH,), jnp.float32),
        pltpu.VMEM((RCH,), jnp.float32),
        pltpu.SemaphoreType.DMA,
    ],
)(_k1_body)


def _k2_body(src_h, dst_h, dis_h, hls_h,
             hg_h, a_h, b_h,
             acc_sh, disf, isrc, idst, ixw, xrow, fin, prev, sem):
    c = lax.axis_index("c")
    s = lax.axis_index("s")

    hops = [(hls_h, a_h), (a_h, b_h), (b_h, hg_h)]
    for hop, (src_ref, dst_ref) in enumerate(hops):
        _zero_buf(fin, RCH, NH)
        for k in range(NRC):
            pltpu.sync_copy(fin, acc_sh.at[pl.ds(s * RPT + k * RCH, RCH)])
        plsc.subcore_barrier()

        def edge_chunk(g, carry):
            base = s * EPT + g * C
            pltpu.sync_copy(src_h.at[pl.ds(base, C)], isrc)
            pltpu.sync_copy(dst_h.at[pl.ds(base, C)], idst)
            for grp in range(C // L):
                dg = idst[pl.ds(grp * L, L)]
                ixw[pl.ds(grp * L, L)] = dg + c * NP
            pltpu.async_copy(src_ref.at[ixw], xrow, sem).wait()
            pltpu.sync_copy(xrow, acc_sh.at[isrc], add=True)
            return carry
        lax.fori_loop(0, NCHUNK, edge_chunk, 0)
        plsc.subcore_barrier()

        for k in range(NRC):
            rb = s * RPT + k * RCH
            pltpu.sync_copy(acc_sh.at[pl.ds(rb, RCH)], fin)
            pltpu.sync_copy(src_ref.at[pl.ds(c * NP + rb, RCH)], prev)
            pltpu.sync_copy(dis_h.at[pl.ds(rb, RCH)], disf.at[pl.ds(0, RCH)])

            def fin_row(i, carry):
                di = disf[pl.ds(i, L)][0]
                sw = di * di if hop < K - 1 else di
                for j in range(NH // L):
                    fin[i, pl.ds(j * L, L)] = (fin[i, pl.ds(j * L, L)] +
                                               prev[i, pl.ds(j * L, L)]) * sw
                return carry
            lax.fori_loop(0, RCH, fin_row, 0)
            pltpu.sync_copy(fin, dst_ref.at[pl.ds(c * NP + rb, RCH)])
        plsc.subcore_barrier()


_k2 = functools.partial(
    pl.kernel,
    out_type=(jax.ShapeDtypeStruct((HEADS * NP, NH), jnp.float32),
              jax.ShapeDtypeStruct((HEADS * NP, NH), jnp.float32),
              jax.ShapeDtypeStruct((HEADS * NP, NH), jnp.float32)),
    mesh=_MESH,
    compiler_params=pltpu.CompilerParams(needs_layout_passes=False),
    scratch_types=[
        pltpu.VMEM_SHARED((NP, NH), jnp.float32),
        pltpu.VMEM((RCH + L,), jnp.float32),
        pltpu.VMEM((C,), jnp.int32),
        pltpu.VMEM((C,), jnp.int32),
        pltpu.VMEM((C,), jnp.int32),
        pltpu.VMEM((C, NH), jnp.float32),
        pltpu.VMEM((RCH, NH), jnp.float32),
        pltpu.VMEM((RCH, NH), jnp.float32),
        pltpu.SemaphoreType.DMA,
    ],
)(_k2_body)


def _fuse_body(hl_ref, hg_ref, w_ref, b_ref, out_ref):
    acc = b_ref[...]
    acc = acc + _dot(hl_ref[0], w_ref[0:NH, :], ((1,), (0,)))
    acc = acc + _dot(hl_ref[1], w_ref[NH:2 * NH, :], ((1,), (0,)))
    acc = acc + _dot(hg_ref[0], w_ref[2 * NH:3 * NH, :], ((1,), (0,)))
    acc = acc + _dot(hg_ref[1], w_ref[3 * NH:4 * NH, :], ((1,), (0,)))
    out_ref[...] = jnp.where(acc > 0.0, acc, jnp.exp(acc) - 1.0)


def _fuse(hl, hg, fw, fb):
    grid = NP // _BR
    return pl.pallas_call(
        _fuse_body,
        grid=(grid,),
        in_specs=[
            pl.BlockSpec((HEADS, _BR, NH), lambda i: (0, i, 0)),
            pl.BlockSpec((HEADS, _BR, NH), lambda i: (0, i, 0)),
            pl.BlockSpec((2 * HEADS * NH, HEADS * NH), lambda i: (0, 0)),
            pl.BlockSpec((1, HEADS * NH), lambda i: (0, 0)),
        ],
        out_specs=pl.BlockSpec((_BR, HEADS * NH), lambda i: (i, 0)),
        out_shape=jax.ShapeDtypeStruct((NP, HEADS * NH), jnp.float32),
    )(hl, hg, fw, fb)


@jax.jit
def kernel(entity_embeddings, relation_embeddings, edge_list, edge_type,
           thW1, thb1, thW2, thb2, c_r, W_att, W_r, a_att, W_out,
           fuse_W, fuse_b):
    pad = jnp.full((EPAD - E,), NP - 1, jnp.int32)
    src = jnp.concatenate([edge_list[0], pad])
    dst = jnp.concatenate([edge_list[1], pad])
    et = jnp.concatenate([edge_type, jnp.zeros((EPAD - E,), jnp.int32)])
    x_pad = jnp.pad(entity_embeddings, ((0, NP - N), (0, 0)))
    t_tab, s3_tab, xw_tab, x_rel = _prep(
        x_pad, relation_embeddings,
        thW1, thb1.reshape(HEADS, 1, 512), thW2, thb2.reshape(HEADS, 1, NF),
        c_r, W_att, W_r, a_att, W_out)
    s3_tab = s3_tab.reshape(HEADS * NP)
    t_tab = t_tab.reshape(HEADS * NP * R)
    xw_tab = xw_tab.reshape(HEADS * NP, NH)
    hl, hls, dis = _k1(src, dst, et, t_tab, s3_tab, xw_tab)
    hg, _, _ = _k2(src, dst, dis, hls)
    h_fused = _fuse(hl.reshape(HEADS, NP, NH), hg.reshape(HEADS, NP, NH),
                    fuse_W, fuse_b.reshape(1, HEADS * NH))
    return (h_fused[:N], x_rel)


# speedup vs baseline: 2.2556x; 2.2556x over previous
import functools

import jax
import jax.numpy as jnp
from jax import lax
from jax.experimental import pallas as pl
from jax.experimental.pallas import tpu as pltpu
from jax.experimental.pallas import tpu_sc as plsc

N = 10000
E = 320000
NF = 128
NH = 128
HEADS = 2
R = 16
ALPHA = 0.2
K = 3

NP = 10240
NC = 2
NS = 16
L = 16
C = 128
EPT = 20480
EPAD = NS * EPT
NCHUNK = EPT // C
C2 = 80
NCH2 = EPT // C2
RPT = NP // NS
RCH = 64
NRC = RPT // RCH
AC = NH + L


_BR = 1024


def _dot(a, b, dims):
    return lax.dot_general(a, b, (dims, ((), ())),
                           preferred_element_type=jnp.float32)


def _prep_body(x_ref, rel_ref, w1_ref, b1_ref, w2_ref, b2_ref, cr_ref,
               wa_ref, wr_ref, aa_ref, wo_ref,
               t_ref, s3_ref, xw_ref, xrel_ref):
    xb = x_ref[...]
    rel = rel_ref[...]
    for h in range(HEADS):
        hid = jnp.maximum(_dot(rel, w1_ref[h], ((1,), (0,))) + b1_ref[h], 0.0)
        theta = _dot(hid, w2_ref[h], ((1,), (0,))) + b2_ref[h]
        ct = jnp.cos(theta)
        a = aa_ref[h]
        a1 = a[:, 0:NH]
        a2 = a[:, NH:2 * NH]
        a3 = a[:, 2 * NH:3 * NH]
        v1 = _dot(a1, wa_ref[h], ((1,), (1,)))
        v2 = _dot(a2, wr_ref[h], ((1,), (1,)))
        v3 = _dot(a3, wa_ref[h], ((1,), (1,)))
        s1 = _dot(xb, v1, ((1,), (1,)))
        s2 = _dot(xb, ct * v2, ((1,), (1,)))
        cterm = _dot(cr_ref[h], v2, ((1,), (1,)))
        t_ref[h] = s1 + s2 + cterm
        s3_ref[h] = _dot(xb, v3, ((1,), (1,)))
        xw_ref[h] = _dot(xb, wa_ref[h], ((1,), (0,)))
    xrel_ref[...] = _dot(rel, wo_ref[...], ((1,), (0,)))


def _prep(x_pad, rel, w1, b1, w2, b2, cr, wa, wr, aa, wo):
    full = lambda shape: pl.BlockSpec(shape, lambda i: tuple(0 for _ in shape))
    grid = NP // _BR
    return pl.pallas_call(
        _prep_body,
        grid=(grid,),
        in_specs=[
            pl.BlockSpec((_BR, NF), lambda i: (i, 0)),
            full((R, NF)),
            full((HEADS, NF, 512)),
            full((HEADS, 1, 512)),
            full((HEADS, 512, NF)),
            full((HEADS, 1, NF)),
            full((HEADS, 1, NF)),
            full((HEADS, NF, NH)),
            full((HEADS, NF, NH)),
            full((HEADS, 1, 3 * NH)),
            full((NF, HEADS * NH)),
        ],
        out_specs=[
            pl.BlockSpec((HEADS, _BR, R), lambda i: (0, i, 0)),
            pl.BlockSpec((HEADS, _BR, 1), lambda i: (0, i, 0)),
            pl.BlockSpec((HEADS, _BR, NH), lambda i: (0, i, 0)),
            pl.BlockSpec((R, HEADS * NH), lambda i: (0, 0)),
        ],
        out_shape=[
            jax.ShapeDtypeStruct((HEADS, NP, R), jnp.float32),
            jax.ShapeDtypeStruct((HEADS, NP, 1), jnp.float32),
            jax.ShapeDtypeStruct((HEADS, NP, NH), jnp.float32),
            jax.ShapeDtypeStruct((R, HEADS * NH), jnp.float32),
        ],
    )(x_pad, rel, w1, b1, w2, b2, cr, wa, wr, aa, wo)


_MESH = plsc.VectorSubcoreMesh(core_axis_name="c", subcore_axis_name="s")


def _zero_buf(buf, rows, cols):
    def zrow(i, carry):
        for j in range(cols // L):
            buf[i, pl.ds(j * L, L)] = jnp.zeros((L,), jnp.float32)
        return carry
    lax.fori_loop(0, rows, zrow, 0)


def _k1_body(src_h, dst_h, et_h, t_h, s3_h, xw_h,
             hl_h, hls_h, dis_h,
             acc_sh, rsum_sh, deg_sh,
             s3_v, isrc, idst, iet, iflat, ixw, tval, xrow, e_v, ones_v,
             fin, disc, invc, rs_v, deg_v, sem):
    c = lax.axis_index("c")
    s = lax.axis_index("s")
    lane = lax.iota(jnp.int32, L)

    _zero_buf(fin, RCH, NH)
    for gg in range(RCH // L):
        rs_v[pl.ds(gg * L, L)] = jnp.zeros((L,), jnp.float32)
    for grp in range(C // L):
        ones_v[pl.ds(grp * L, L)] = jnp.ones((L,), jnp.float32)
    for k in range(NRC):
        rb = s * RPT + k * RCH
        pltpu.sync_copy(fin, acc_sh.at[pl.ds(rb, RCH)])
        pltpu.sync_copy(rs_v, rsum_sh.at[pl.ds(rb, RCH)])
        pltpu.sync_copy(rs_v, deg_sh.at[pl.ds(rb, RCH)])
    pltpu.sync_copy(s3_h.at[pl.ds(c * NP, NP)], s3_v)
    plsc.subcore_barrier()

    def edge_chunk(g, carry):
        base = s * EPT + g * C
        pltpu.sync_copy(src_h.at[pl.ds(base, C)], isrc)
        pltpu.sync_copy(dst_h.at[pl.ds(base, C)], idst)
        pltpu.sync_copy(et_h.at[pl.ds(base, C)], iet)
        for grp in range(C // L):
            sg = isrc[pl.ds(grp * L, L)]
            etg = iet[pl.ds(grp * L, L)]
            dg = idst[pl.ds(grp * L, L)]
            iflat[pl.ds(grp * L, L)] = (sg + c * NP) * R + etg
            ixw[pl.ds(grp * L, L)] = dg + c * NP
        pltpu.async_copy(t_h.at[iflat], tval, sem).wait()
        pltpu.async_copy(xw_h.at[ixw], xrow, sem).wait()
        for grp in range(C // L):
            dstg = idst[pl.ds(grp * L, L)]
            tv = tval[pl.ds(grp * L, L)]
            s3d = plsc.load_gather(s3_v, [dstg])
            logit = tv + s3d
            lr = jnp.where(logit >= 0.0, logit, ALPHA * logit)
            e_v[pl.ds(grp * L, L)] = jnp.exp(-lr)

        def scale_row(i, carry2):
            ei = e_v[pl.ds(i, L)][0]
            for j in range(NH // L):
                xrow[i, pl.ds(j * L, L)] = xrow[i, pl.ds(j * L, L)] * ei
            return carry2
        lax.fori_loop(0, C, scale_row, 0)
        pltpu.sync_copy(xrow, acc_sh.at[isrc], add=True)
        pltpu.sync_copy(e_v.at[pl.ds(0, C)], rsum_sh.at[isrc], add=True)
        pltpu.sync_copy(ones_v, deg_sh.at[isrc], add=True)
        return carry
    lax.fori_loop(0, NCHUNK, edge_chunk, 0)
    plsc.subcore_barrier()

    for k in range(NRC):
        rb = s * RPT + k * RCH
        pltpu.sync_copy(acc_sh.at[pl.ds(rb, RCH)], fin)
        pltpu.sync_copy(rsum_sh.at[pl.ds(rb, RCH)], rs_v)
        pltpu.sync_copy(deg_sh.at[pl.ds(rb, RCH)], deg_v)
        for gg in range(RCH // L):
            rs = rs_v[pl.ds(gg * L, L)]
            rs = jnp.where(rs == 0.0, 1e-12, rs)
            degv = deg_v[pl.ds(gg * L, L)] + 1.0
            ii = plsc.bitcast(degv, jnp.int32)
            ii = jnp.int32(0x5F3759DF) - lax.shift_right_arithmetic(ii, 1)
            y = plsc.bitcast(ii, jnp.float32)
            for _ in range(3):
                y = y * (1.5 - 0.5 * degv * y * y)
            invc[pl.ds(gg * L, L)] = 1.0 / rs
            disc[pl.ds(gg * L, L)] = y

        def fin_row(i, carry):
            iv = invc[pl.ds(i, L)][0]
            for j in range(NH // L):
                v = fin[i, pl.ds(j * L, L)] * iv
                fin[i, pl.ds(j * L, L)] = jnp.where(v > 0.0, v,
                                                    jnp.exp(v) - 1.0)
            return carry
        lax.fori_loop(0, RCH, fin_row, 0)
        pltpu.sync_copy(fin, hl_h.at[pl.ds(c * NP + rb, RCH)])

        def scl_row(i, carry):
            di = disc[pl.ds(i, L)][0]
            for j in range(NH // L):
                fin[i, pl.ds(j * L, L)] = fin[i, pl.ds(j * L, L)] * di
            return carry
        lax.fori_loop(0, RCH, scl_row, 0)
        pltpu.sync_copy(fin, hls_h.at[pl.ds(c * NP + rb, RCH)])

        @pl.when(c == 0)
        def _():
            pltpu.sync_copy(disc.at[pl.ds(0, RCH)], dis_h.at[pl.ds(rb, RCH)])


_k1 = functools.partial(
    pl.kernel,
    out_type=(jax.ShapeDtypeStruct((HEADS * NP, NH), jnp.float32),
              jax.ShapeDtypeStruct((HEADS * NP, NH), jnp.float32),
              jax.ShapeDtypeStruct((NP,), jnp.float32)),
    mesh=_MESH,
    compiler_params=pltpu.CompilerParams(needs_layout_passes=False),
    scratch_types=[
        pltpu.VMEM_SHARED((NP, NH), jnp.float32),
        pltpu.VMEM_SHARED((NP,), jnp.float32),
        pltpu.VMEM_SHARED((NP,), jnp.float32),
        pltpu.VMEM((NP,), jnp.float32),
        pltpu.VMEM((C,), jnp.int32),
        pltpu.VMEM((C,), jnp.int32),
        pltpu.VMEM((C,), jnp.int32),
        pltpu.VMEM((C,), jnp.int32),
        pltpu.VMEM((C,), jnp.int32),
        pltpu.VMEM((C,), jnp.float32),
        pltpu.VMEM((C, NH), jnp.float32),
        pltpu.VMEM((C + L,), jnp.float32),
        pltpu.VMEM((C,), jnp.float32),
        pltpu.VMEM((RCH, NH), jnp.float32),
        pltpu.VMEM((RCH + L,), jnp.float32),
        pltpu.VMEM((RCH + L,), jnp.float32),
        pltpu.VMEM((RCH,), jnp.float32),
        pltpu.VMEM((RCH,), jnp.float32),
        pltpu.SemaphoreType.DMA,
    ],
)(_k1_body)


def _k2_body(src_h, dst_h, dis_h, hls_h,
             hg_h, a_h, b_h,
             acc_sh, disf, isrc4, idst4, ixw4, xra, xrb, fin, prev,
             si0, si1, si2, si3, sg0, sg1, ss0, ss1):
    c = lax.axis_index("c")
    s = lax.axis_index("s")
    semi = [si0, si1, si2, si3]
    semg = [sg0, sg1]
    sems = [ss0, ss1]
    xr = [xra, xrb]

    def issue_idx(g, j):
        base = s * EPT + g * C2
        pltpu.async_copy(src_h.at[pl.ds(base, C2)], isrc4.at[j], semi[j])
        pltpu.async_copy(dst_h.at[pl.ds(base, C2)], idst4.at[j], semi[j])

    def wait_idx(j):
        pltpu.make_async_copy(src_h.at[pl.ds(0, C2)], isrc4.at[j],
                              semi[j]).wait()
        pltpu.make_async_copy(dst_h.at[pl.ds(0, C2)], idst4.at[j],
                              semi[j]).wait()

    def comp_ixw(j):
        for grp in range(C2 // L):
            dg = idst4[j, pl.ds(grp * L, L)]
            ixw4[j, pl.ds(grp * L, L)] = dg + c * NP

    hops = [(hls_h, a_h), (a_h, b_h), (b_h, hg_h)]
    for hop, (src_ref, dst_ref) in enumerate(hops):
        _zero_buf(fin, RCH, NH)
        for k in range(NRC):
            pltpu.sync_copy(fin, acc_sh.at[pl.ds(s * RPT + k * RCH, RCH)])
        plsc.subcore_barrier()

        def issue_gather(j, p):
            pltpu.async_copy(src_ref.at[ixw4.at[j]], xr[p], semg[p])

        def wait_gather(p):
            pltpu.make_async_copy(src_ref.at[pl.ds(0, C2)], xr[p],
                                  semg[p]).wait()

        def issue_scatter(j, p):
            pltpu.async_copy(xr[p], acc_sh.at[isrc4.at[j]], sems[p], add=True)

        def wait_scatter(p):
            pltpu.make_async_copy(src_ref.at[pl.ds(0, C2)], xr[p],
                                  sems[p]).wait()

        def chunk_step(g, r, have_g1, have_g3, have_gm1):
            j, j1, j3 = r % 4, (r + 1) % 4, (r + 3) % 4
            p, p1 = r % 2, (r + 1) % 2
            if have_g1:
                wait_idx(j1)
                comp_ixw(j1)
            if have_gm1:
                wait_scatter(p1)
            if have_g3:
                issue_idx(g + 3, j3)
            if have_g1:
                issue_gather(j1, p1)
            wait_gather(p)
            issue_scatter(j, p)

        issue_idx(0, 0)
        issue_idx(1, 1)
        issue_idx(2, 2)
        wait_idx(0)
        comp_ixw(0)
        issue_gather(0, 0)
        for r in range(4):
            chunk_step(r, r, True, True, r >= 1)

        def quad(q, carry):
            for r in range(4):
                chunk_step(q * 4 + r, r, True, True, True)
            return carry
        lax.fori_loop(1, NCH2 // 4 - 1, quad, 0)

        for r in range(4):
            g = NCH2 - 4 + r
            chunk_step(g, r, r < 3, r < 1, True)
        wait_scatter(1)
        plsc.subcore_barrier()

        for k in range(NRC):
            rb = s * RPT + k * RCH
            pltpu.sync_copy(acc_sh.at[pl.ds(rb, RCH)], fin)
            pltpu.sync_copy(src_ref.at[pl.ds(c * NP + rb, RCH)], prev)
            pltpu.sync_copy(dis_h.at[pl.ds(rb, RCH)], disf.at[pl.ds(0, RCH)])

            def fin_row(i, carry):
                di = disf[pl.ds(i, L)][0]
                sw = di * di if hop < K - 1 else di
                for j in range(NH // L):
                    fin[i, pl.ds(j * L, L)] = (fin[i, pl.ds(j * L, L)] +
                                               prev[i, pl.ds(j * L, L)]) * sw
                return carry
            lax.fori_loop(0, RCH, fin_row, 0)
            pltpu.sync_copy(fin, dst_ref.at[pl.ds(c * NP + rb, RCH)])
        plsc.subcore_barrier()


_k2 = functools.partial(
    pl.kernel,
    out_type=(jax.ShapeDtypeStruct((HEADS * NP, NH), jnp.float32),
              jax.ShapeDtypeStruct((HEADS * NP, NH), jnp.float32),
              jax.ShapeDtypeStruct((HEADS * NP, NH), jnp.float32)),
    mesh=_MESH,
    compiler_params=pltpu.CompilerParams(needs_layout_passes=False),
    scratch_types=[
        pltpu.VMEM_SHARED((NP, NH), jnp.float32),
        pltpu.VMEM((RCH + L,), jnp.float32),
        pltpu.VMEM((4, C2), jnp.int32),
        pltpu.VMEM((4, C2), jnp.int32),
        pltpu.VMEM((4, C2), jnp.int32),
        pltpu.VMEM((C2, NH), jnp.float32),
        pltpu.VMEM((C2, NH), jnp.float32),
        pltpu.VMEM((RCH, NH), jnp.float32),
        pltpu.VMEM((RCH, NH), jnp.float32),
        pltpu.SemaphoreType.DMA,
        pltpu.SemaphoreType.DMA,
        pltpu.SemaphoreType.DMA,
        pltpu.SemaphoreType.DMA,
        pltpu.SemaphoreType.DMA,
        pltpu.SemaphoreType.DMA,
        pltpu.SemaphoreType.DMA,
        pltpu.SemaphoreType.DMA,
    ],
)(_k2_body)


def _fuse_body(hl_ref, hg_ref, w_ref, b_ref, out_ref):
    acc = b_ref[...]
    acc = acc + _dot(hl_ref[0], w_ref[0:NH, :], ((1,), (0,)))
    acc = acc + _dot(hl_ref[1], w_ref[NH:2 * NH, :], ((1,), (0,)))
    acc = acc + _dot(hg_ref[0], w_ref[2 * NH:3 * NH, :], ((1,), (0,)))
    acc = acc + _dot(hg_ref[1], w_ref[3 * NH:4 * NH, :], ((1,), (0,)))
    out_ref[...] = jnp.where(acc > 0.0, acc, jnp.exp(acc) - 1.0)


def _fuse(hl, hg, fw, fb):
    grid = NP // _BR
    return pl.pallas_call(
        _fuse_body,
        grid=(grid,),
        in_specs=[
            pl.BlockSpec((HEADS, _BR, NH), lambda i: (0, i, 0)),
            pl.BlockSpec((HEADS, _BR, NH), lambda i: (0, i, 0)),
            pl.BlockSpec((2 * HEADS * NH, HEADS * NH), lambda i: (0, 0)),
            pl.BlockSpec((1, HEADS * NH), lambda i: (0, 0)),
        ],
        out_specs=pl.BlockSpec((_BR, HEADS * NH), lambda i: (i, 0)),
        out_shape=jax.ShapeDtypeStruct((NP, HEADS * NH), jnp.float32),
    )(hl, hg, fw, fb)


@jax.jit
def kernel(entity_embeddings, relation_embeddings, edge_list, edge_type,
           thW1, thb1, thW2, thb2, c_r, W_att, W_r, a_att, W_out,
           fuse_W, fuse_b):
    pad = N + (jnp.arange(EPAD - E, dtype=jnp.int32) % (NP - N))
    src = jnp.concatenate([edge_list[0], pad])
    dst = jnp.concatenate([edge_list[1], pad])
    et = jnp.concatenate([edge_type, jnp.zeros((EPAD - E,), jnp.int32)])
    x_pad = jnp.pad(entity_embeddings, ((0, NP - N), (0, 0)))
    t_tab, s3_tab, xw_tab, x_rel = _prep(
        x_pad, relation_embeddings,
        thW1, thb1.reshape(HEADS, 1, 512), thW2, thb2.reshape(HEADS, 1, NF),
        c_r, W_att, W_r, a_att, W_out)
    s3_tab = s3_tab.reshape(HEADS * NP)
    t_tab = t_tab.reshape(HEADS * NP * R)
    xw_tab = xw_tab.reshape(HEADS * NP, NH)
    hl, hls, dis = _k1(src, dst, et, t_tab, s3_tab, xw_tab)
    hg, _, _ = _k2(src, dst, dis, hls)
    h_fused = _fuse(hl.reshape(HEADS, NP, NH), hg.reshape(HEADS, NP, NH),
                    fuse_W, fuse_b.reshape(1, HEADS * NH))
    return (h_fused[:N], x_rel)


# speedup vs baseline: 3.3258x; 1.4745x over previous
import functools

import jax
import jax.numpy as jnp
from jax import lax
from jax.experimental import pallas as pl
from jax.experimental.pallas import tpu as pltpu
from jax.experimental.pallas import tpu_sc as plsc

N = 10000
E = 320000
NF = 128
NH = 128
HEADS = 2
R = 16
ALPHA = 0.2
K = 3

NP = 10240
NC = 2
NS = 16
L = 16
C = 80
EPT = 20480
EPAD = NS * EPT
NCHUNK = EPT // C
C2 = C
NCH2 = NCHUNK
RPT = NP // NS
RCH = 64
NRC = RPT // RCH
AC = NH + L


_BR = 1024


def _dot(a, b, dims):
    return lax.dot_general(a, b, (dims, ((), ())),
                           preferred_element_type=jnp.float32)


def _prep_body(x_ref, rel_ref, w1_ref, b1_ref, w2_ref, b2_ref, cr_ref,
               wa_ref, wr_ref, aa_ref, wo_ref,
               t_ref, s3_ref, xw_ref, xrel_ref):
    xb = x_ref[...]
    rel = rel_ref[...]
    for h in range(HEADS):
        hid = jnp.maximum(_dot(rel, w1_ref[h], ((1,), (0,))) + b1_ref[h], 0.0)
        theta = _dot(hid, w2_ref[h], ((1,), (0,))) + b2_ref[h]
        ct = jnp.cos(theta)
        a = aa_ref[h]
        a1 = a[:, 0:NH]
        a2 = a[:, NH:2 * NH]
        a3 = a[:, 2 * NH:3 * NH]
        v1 = _dot(a1, wa_ref[h], ((1,), (1,)))
        v2 = _dot(a2, wr_ref[h], ((1,), (1,)))
        v3 = _dot(a3, wa_ref[h], ((1,), (1,)))
        s1 = _dot(xb, v1, ((1,), (1,)))
        s2 = _dot(xb, ct * v2, ((1,), (1,)))
        cterm = _dot(cr_ref[h], v2, ((1,), (1,)))
        t_ref[h] = s1 + s2 + cterm
        s3_ref[h] = _dot(xb, v3, ((1,), (1,)))
        xw_ref[h] = _dot(xb, wa_ref[h], ((1,), (0,)))
    xrel_ref[...] = _dot(rel, wo_ref[...], ((1,), (0,)))


def _prep(x_pad, rel, w1, b1, w2, b2, cr, wa, wr, aa, wo):
    full = lambda shape: pl.BlockSpec(shape, lambda i: tuple(0 for _ in shape))
    grid = NP // _BR
    return pl.pallas_call(
        _prep_body,
        grid=(grid,),
        in_specs=[
            pl.BlockSpec((_BR, NF), lambda i: (i, 0)),
            full((R, NF)),
            full((HEADS, NF, 512)),
            full((HEADS, 1, 512)),
            full((HEADS, 512, NF)),
            full((HEADS, 1, NF)),
            full((HEADS, 1, NF)),
            full((HEADS, NF, NH)),
            full((HEADS, NF, NH)),
            full((HEADS, 1, 3 * NH)),
            full((NF, HEADS * NH)),
        ],
        out_specs=[
            pl.BlockSpec((HEADS, _BR, R), lambda i: (0, i, 0)),
            pl.BlockSpec((HEADS, _BR, 1), lambda i: (0, i, 0)),
            pl.BlockSpec((HEADS, _BR, NH), lambda i: (0, i, 0)),
            pl.BlockSpec((R, HEADS * NH), lambda i: (0, 0)),
        ],
        out_shape=[
            jax.ShapeDtypeStruct((HEADS, NP, R), jnp.float32),
            jax.ShapeDtypeStruct((HEADS, NP, 1), jnp.float32),
            jax.ShapeDtypeStruct((HEADS, NP, NH), jnp.float32),
            jax.ShapeDtypeStruct((R, HEADS * NH), jnp.float32),
        ],
    )(x_pad, rel, w1, b1, w2, b2, cr, wa, wr, aa, wo)


_MESH = plsc.VectorSubcoreMesh(core_axis_name="c", subcore_axis_name="s")


def _zero_buf(buf, rows, cols):
    def zrow(i, carry):
        for j in range(cols // L):
            buf[i, pl.ds(j * L, L)] = jnp.zeros((L,), jnp.float32)
        return carry
    lax.fori_loop(0, rows, zrow, 0)


def _k1_body(src_h, dst_h, et_h, t_h, s3_h, xw_h,
             hl_h, hls_h, dis_h,
             acc_sh, rsum_sh, deg_sh,
             isrc4, idst4, iet4, iflat4, ixw4, tval2, s3val2, xra, xrb,
             e_a, e_b, ones_v, fin, disc, invc, rs_v, deg_v,
             si0, si1, si2, si3, st0, st1, sg0, sg1, ss0, ss1):
    c = lax.axis_index("c")
    s = lax.axis_index("s")
    semi = [si0, si1, si2, si3]
    semt = [st0, st1]
    semg = [sg0, sg1]
    sems = [ss0, ss1]
    xr = [xra, xrb]
    ev = [e_a, e_b]

    _zero_buf(fin, RCH, NH)
    for gg in range(RCH // L):
        rs_v[pl.ds(gg * L, L)] = jnp.zeros((L,), jnp.float32)
    for grp in range(C // L):
        ones_v[pl.ds(grp * L, L)] = jnp.ones((L,), jnp.float32)
    for k in range(NRC):
        rb = s * RPT + k * RCH
        pltpu.sync_copy(fin, acc_sh.at[pl.ds(rb, RCH)])
        pltpu.sync_copy(rs_v, rsum_sh.at[pl.ds(rb, RCH)])
        pltpu.sync_copy(rs_v, deg_sh.at[pl.ds(rb, RCH)])
    plsc.subcore_barrier()

    def issue_idx(g, j):
        base = s * EPT + g * C
        pltpu.async_copy(src_h.at[pl.ds(base, C)], isrc4.at[j], semi[j])
        pltpu.async_copy(dst_h.at[pl.ds(base, C)], idst4.at[j], semi[j])
        pltpu.async_copy(et_h.at[pl.ds(base, C)], iet4.at[j], semi[j])

    def wait_idx(j):
        pltpu.make_async_copy(src_h.at[pl.ds(0, C)], isrc4.at[j],
                              semi[j]).wait()
        pltpu.make_async_copy(dst_h.at[pl.ds(0, C)], idst4.at[j],
                              semi[j]).wait()
        pltpu.make_async_copy(et_h.at[pl.ds(0, C)], iet4.at[j],
                              semi[j]).wait()

    def comp_idx(j):
        for grp in range(C // L):
            sg = isrc4[j, pl.ds(grp * L, L)]
            etg = iet4[j, pl.ds(grp * L, L)]
            dg = idst4[j, pl.ds(grp * L, L)]
            iflat4[j, pl.ds(grp * L, L)] = (sg + c * NP) * R + etg
            ixw4[j, pl.ds(grp * L, L)] = dg + c * NP

    def issue_gathers(j, p):
        pltpu.async_copy(t_h.at[iflat4.at[j]], tval2.at[p], semt[p])
        pltpu.async_copy(s3_h.at[ixw4.at[j]], s3val2.at[p], semt[p])
        pltpu.async_copy(xw_h.at[ixw4.at[j]], xr[p], semg[p])

    def wait_gathers(p):
        pltpu.make_async_copy(t_h.at[pl.ds(0, C)], tval2.at[p],
                              semt[p]).wait()
        pltpu.make_async_copy(s3_h.at[pl.ds(0, C)], s3val2.at[p],
                              semt[p]).wait()
        pltpu.make_async_copy(xw_h.at[pl.ds(0, C)], xr[p], semg[p]).wait()

    def compute_e(p):
        for grp in range(C // L):
            tv = tval2[p, pl.ds(grp * L, L)]
            s3d = s3val2[p, pl.ds(grp * L, L)]
            logit = tv + s3d
            lr = jnp.where(logit >= 0.0, logit, ALPHA * logit)
            ev[p][pl.ds(grp * L, L)] = jnp.exp(-lr)

    def scale(p):
        xrp = xr[p]
        evp = ev[p]

        def scale_row(i, carry2):
            ei = evp[pl.ds(i, L)][0]
            for j in range(NH // L):
                xrp[i, pl.ds(j * L, L)] = xrp[i, pl.ds(j * L, L)] * ei
            return carry2
        lax.fori_loop(0, C, scale_row, 0)

    def issue_scatters(j, p):
        pltpu.async_copy(xr[p], acc_sh.at[isrc4.at[j]], sems[p], add=True)
        pltpu.async_copy(ev[p].at[pl.ds(0, C)], rsum_sh.at[isrc4.at[j]],
                         sems[p], add=True)
        pltpu.async_copy(ones_v, deg_sh.at[isrc4.at[j]], sems[p], add=True)

    def wait_scatters(p):
        pltpu.make_async_copy(xw_h.at[pl.ds(0, C)], xr[p], sems[p]).wait()
        pltpu.make_async_copy(t_h.at[pl.ds(0, C)], ev[p].at[pl.ds(0, C)],
                              sems[p]).wait()
        pltpu.make_async_copy(t_h.at[pl.ds(0, C)], ones_v, sems[p]).wait()

    def chunk_step(g, r, have_g1, have_g3, have_gm1):
        j, j1, j3 = r % 4, (r + 1) % 4, (r + 3) % 4
        p, p1 = r % 2, (r + 1) % 2
        if have_g1:
            wait_idx(j1)
            comp_idx(j1)
        if have_gm1:
            wait_scatters(p1)
        if have_g3:
            issue_idx(g + 3, j3)
        if have_g1:
            issue_gathers(j1, p1)
        wait_gathers(p)
        compute_e(p)
        scale(p)
        issue_scatters(j, p)

    issue_idx(0, 0)
    issue_idx(1, 1)
    issue_idx(2, 2)
    wait_idx(0)
    comp_idx(0)
    issue_gathers(0, 0)
    for r in range(4):
        chunk_step(r, r, True, True, r >= 1)

    def quad(q, carry):
        for r in range(4):
            chunk_step(q * 4 + r, r, True, True, True)
        return carry
    lax.fori_loop(1, NCHUNK // 4 - 1, quad, 0)

    for r in range(4):
        chunk_step(NCHUNK - 4 + r, r, r < 3, r < 1, True)
    wait_scatters(1)
    plsc.subcore_barrier()

    for k in range(NRC):
        rb = s * RPT + k * RCH
        pltpu.sync_copy(acc_sh.at[pl.ds(rb, RCH)], fin)
        pltpu.sync_copy(rsum_sh.at[pl.ds(rb, RCH)], rs_v)
        pltpu.sync_copy(deg_sh.at[pl.ds(rb, RCH)], deg_v)
        for gg in range(RCH // L):
            rs = rs_v[pl.ds(gg * L, L)]
            rs = jnp.where(rs == 0.0, 1e-12, rs)
            degv = deg_v[pl.ds(gg * L, L)] + 1.0
            ii = plsc.bitcast(degv, jnp.int32)
            ii = jnp.int32(0x5F3759DF) - lax.shift_right_arithmetic(ii, 1)
            y = plsc.bitcast(ii, jnp.float32)
            for _ in range(3):
                y = y * (1.5 - 0.5 * degv * y * y)
            invc[pl.ds(gg * L, L)] = 1.0 / rs
            disc[pl.ds(gg * L, L)] = y

        def fin_row(i, carry):
            iv = invc[pl.ds(i, L)][0]
            for j in range(NH // L):
                v = fin[i, pl.ds(j * L, L)] * iv
                fin[i, pl.ds(j * L, L)] = jnp.where(v > 0.0, v,
                                                    jnp.exp(v) - 1.0)
            return carry
        lax.fori_loop(0, RCH, fin_row, 0)
        pltpu.sync_copy(fin, hl_h.at[pl.ds(c * NP + rb, RCH)])

        def scl_row(i, carry):
            di = disc[pl.ds(i, L)][0]
            for j in range(NH // L):
                fin[i, pl.ds(j * L, L)] = fin[i, pl.ds(j * L, L)] * di
            return carry
        lax.fori_loop(0, RCH, scl_row, 0)
        pltpu.sync_copy(fin, hls_h.at[pl.ds(c * NP + rb, RCH)])

        @pl.when(c == 0)
        def _():
            pltpu.sync_copy(disc.at[pl.ds(0, RCH)], dis_h.at[pl.ds(rb, RCH)])


_k1 = functools.partial(
    pl.kernel,
    out_type=(jax.ShapeDtypeStruct((HEADS * NP, NH), jnp.float32),
              jax.ShapeDtypeStruct((HEADS * NP, NH), jnp.float32),
              jax.ShapeDtypeStruct((NP,), jnp.float32)),
    mesh=_MESH,
    compiler_params=pltpu.CompilerParams(needs_layout_passes=False),
    scratch_types=[
        pltpu.VMEM_SHARED((NP, NH), jnp.float32),
        pltpu.VMEM_SHARED((NP,), jnp.float32),
        pltpu.VMEM_SHARED((NP,), jnp.float32),
        pltpu.VMEM((4, C), jnp.int32),
        pltpu.VMEM((4, C), jnp.int32),
        pltpu.VMEM((4, C), jnp.int32),
        pltpu.VMEM((4, C), jnp.int32),
        pltpu.VMEM((4, C), jnp.int32),
        pltpu.VMEM((2, C), jnp.float32),
        pltpu.VMEM((2, C), jnp.float32),
        pltpu.VMEM((C, NH), jnp.float32),
        pltpu.VMEM((C, NH), jnp.float32),
        pltpu.VMEM((C + L,), jnp.float32),
        pltpu.VMEM((C + L,), jnp.float32),
        pltpu.VMEM((C,), jnp.float32),
        pltpu.VMEM((RCH, NH), jnp.float32),
        pltpu.VMEM((RCH + L,), jnp.float32),
        pltpu.VMEM((RCH + L,), jnp.float32),
        pltpu.VMEM((RCH,), jnp.float32),
        pltpu.VMEM((RCH,), jnp.float32),
        pltpu.SemaphoreType.DMA,
        pltpu.SemaphoreType.DMA,
        pltpu.SemaphoreType.DMA,
        pltpu.SemaphoreType.DMA,
        pltpu.SemaphoreType.DMA,
        pltpu.SemaphoreType.DMA,
        pltpu.SemaphoreType.DMA,
        pltpu.SemaphoreType.DMA,
        pltpu.SemaphoreType.DMA,
        pltpu.SemaphoreType.DMA,
    ],
)(_k1_body)


def _k2_body(src_h, dst_h, dis_h, hls_h,
             hg_h, a_h, b_h,
             acc_sh, disf, isrc4, idst4, ixw4, xra, xrb, fin, prev,
             si0, si1, si2, si3, sg0, sg1, ss0, ss1):
    c = lax.axis_index("c")
    s = lax.axis_index("s")
    semi = [si0, si1, si2, si3]
    semg = [sg0, sg1]
    sems = [ss0, ss1]
    xr = [xra, xrb]

    def issue_idx(g, j):
        base = s * EPT + g * C2
        pltpu.async_copy(src_h.at[pl.ds(base, C2)], isrc4.at[j], semi[j])
        pltpu.async_copy(dst_h.at[pl.ds(base, C2)], idst4.at[j], semi[j])

    def wait_idx(j):
        pltpu.make_async_copy(src_h.at[pl.ds(0, C2)], isrc4.at[j],
                              semi[j]).wait()
        pltpu.make_async_copy(dst_h.at[pl.ds(0, C2)], idst4.at[j],
                              semi[j]).wait()

    def comp_ixw(j):
        for grp in range(C2 // L):
            dg = idst4[j, pl.ds(grp * L, L)]
            ixw4[j, pl.ds(grp * L, L)] = dg + c * NP

    hops = [(hls_h, a_h), (a_h, b_h), (b_h, hg_h)]
    for hop, (src_ref, dst_ref) in enumerate(hops):
        _zero_buf(fin, RCH, NH)
        for k in range(NRC):
            pltpu.sync_copy(fin, acc_sh.at[pl.ds(s * RPT + k * RCH, RCH)])
        plsc.subcore_barrier()

        def issue_gather(j, p):
            pltpu.async_copy(src_ref.at[ixw4.at[j]], xr[p], semg[p])

        def wait_gather(p):
            pltpu.make_async_copy(src_ref.at[pl.ds(0, C2)], xr[p],
                                  semg[p]).wait()

        def issue_scatter(j, p):
            pltpu.async_copy(xr[p], acc_sh.at[isrc4.at[j]], sems[p], add=True)

        def wait_scatter(p):
            pltpu.make_async_copy(src_ref.at[pl.ds(0, C2)], xr[p],
                                  sems[p]).wait()

        def chunk_step(g, r, have_g1, have_g3, have_gm1):
            j, j1, j3 = r % 4, (r + 1) % 4, (r + 3) % 4
            p, p1 = r % 2, (r + 1) % 2
            if have_g1:
                wait_idx(j1)
                comp_ixw(j1)
            if have_gm1:
                wait_scatter(p1)
            if have_g3:
                issue_idx(g + 3, j3)
            if have_g1:
                issue_gather(j1, p1)
            wait_gather(p)
            issue_scatter(j, p)

        issue_idx(0, 0)
        issue_idx(1, 1)
        issue_idx(2, 2)
        wait_idx(0)
        comp_ixw(0)
        issue_gather(0, 0)
        for r in range(4):
            chunk_step(r, r, True, True, r >= 1)

        def quad(q, carry):
            for r in range(4):
                chunk_step(q * 4 + r, r, True, True, True)
            return carry
        lax.fori_loop(1, NCH2 // 4 - 1, quad, 0)

        for r in range(4):
            g = NCH2 - 4 + r
            chunk_step(g, r, r < 3, r < 1, True)
        wait_scatter(1)
        plsc.subcore_barrier()

        for k in range(NRC):
            rb = s * RPT + k * RCH
            pltpu.sync_copy(acc_sh.at[pl.ds(rb, RCH)], fin)
            pltpu.sync_copy(src_ref.at[pl.ds(c * NP + rb, RCH)], prev)
            pltpu.sync_copy(dis_h.at[pl.ds(rb, RCH)], disf.at[pl.ds(0, RCH)])

            def fin_row(i, carry):
                di = disf[pl.ds(i, L)][0]
                sw = di * di if hop < K - 1 else di
                for j in range(NH // L):
                    fin[i, pl.ds(j * L, L)] = (fin[i, pl.ds(j * L, L)] +
                                               prev[i, pl.ds(j * L, L)]) * sw
                return carry
            lax.fori_loop(0, RCH, fin_row, 0)
            pltpu.sync_copy(fin, dst_ref.at[pl.ds(c * NP + rb, RCH)])
        plsc.subcore_barrier()


_k2 = functools.partial(
    pl.kernel,
    out_type=(jax.ShapeDtypeStruct((HEADS * NP, NH), jnp.float32),
              jax.ShapeDtypeStruct((HEADS * NP, NH), jnp.float32),
              jax.ShapeDtypeStruct((HEADS * NP, NH), jnp.float32)),
    mesh=_MESH,
    compiler_params=pltpu.CompilerParams(needs_layout_passes=False),
    scratch_types=[
        pltpu.VMEM_SHARED((NP, NH), jnp.float32),
        pltpu.VMEM((RCH + L,), jnp.float32),
        pltpu.VMEM((4, C2), jnp.int32),
        pltpu.VMEM((4, C2), jnp.int32),
        pltpu.VMEM((4, C2), jnp.int32),
        pltpu.VMEM((C2, NH), jnp.float32),
        pltpu.VMEM((C2, NH), jnp.float32),
        pltpu.VMEM((RCH, NH), jnp.float32),
        pltpu.VMEM((RCH, NH), jnp.float32),
        pltpu.SemaphoreType.DMA,
        pltpu.SemaphoreType.DMA,
        pltpu.SemaphoreType.DMA,
        pltpu.SemaphoreType.DMA,
        pltpu.SemaphoreType.DMA,
        pltpu.SemaphoreType.DMA,
        pltpu.SemaphoreType.DMA,
        pltpu.SemaphoreType.DMA,
    ],
)(_k2_body)


def _fuse_body(hl_ref, hg_ref, w_ref, b_ref, out_ref):
    acc = b_ref[...]
    acc = acc + _dot(hl_ref[0], w_ref[0:NH, :], ((1,), (0,)))
    acc = acc + _dot(hl_ref[1], w_ref[NH:2 * NH, :], ((1,), (0,)))
    acc = acc + _dot(hg_ref[0], w_ref[2 * NH:3 * NH, :], ((1,), (0,)))
    acc = acc + _dot(hg_ref[1], w_ref[3 * NH:4 * NH, :], ((1,), (0,)))
    out_ref[...] = jnp.where(acc > 0.0, acc, jnp.exp(acc) - 1.0)


def _fuse(hl, hg, fw, fb):
    grid = NP // _BR
    return pl.pallas_call(
        _fuse_body,
        grid=(grid,),
        in_specs=[
            pl.BlockSpec((HEADS, _BR, NH), lambda i: (0, i, 0)),
            pl.BlockSpec((HEADS, _BR, NH), lambda i: (0, i, 0)),
            pl.BlockSpec((2 * HEADS * NH, HEADS * NH), lambda i: (0, 0)),
            pl.BlockSpec((1, HEADS * NH), lambda i: (0, 0)),
        ],
        out_specs=pl.BlockSpec((_BR, HEADS * NH), lambda i: (i, 0)),
        out_shape=jax.ShapeDtypeStruct((NP, HEADS * NH), jnp.float32),
    )(hl, hg, fw, fb)


@jax.jit
def kernel(entity_embeddings, relation_embeddings, edge_list, edge_type,
           thW1, thb1, thW2, thb2, c_r, W_att, W_r, a_att, W_out,
           fuse_W, fuse_b):
    pad = N + (jnp.arange(EPAD - E, dtype=jnp.int32) % (NP - N))
    src = jnp.concatenate([edge_list[0], pad])
    dst = jnp.concatenate([edge_list[1], pad])
    et = jnp.concatenate([edge_type, jnp.zeros((EPAD - E,), jnp.int32)])
    x_pad = jnp.pad(entity_embeddings, ((0, NP - N), (0, 0)))
    t_tab, s3_tab, xw_tab, x_rel = _prep(
        x_pad, relation_embeddings,
        thW1, thb1.reshape(HEADS, 1, 512), thW2, thb2.reshape(HEADS, 1, NF),
        c_r, W_att, W_r, a_att, W_out)
    s3_tab = s3_tab.reshape(HEADS * NP)
    t_tab = t_tab.reshape(HEADS * NP * R)
    xw_tab = xw_tab.reshape(HEADS * NP, NH)
    hl, hls, dis = _k1(src, dst, et, t_tab, s3_tab, xw_tab)
    hg, _, _ = _k2(src, dst, dis, hls)
    h_fused = _fuse(hl.reshape(HEADS, NP, NH), hg.reshape(HEADS, NP, NH),
                    fuse_W, fuse_b.reshape(1, HEADS * NH))
    return (h_fused[:N], x_rel)


# speedup vs baseline: 3.3977x; 1.0216x over previous
import functools

import jax
import jax.numpy as jnp
from jax import lax
from jax.experimental import pallas as pl
from jax.experimental.pallas import tpu as pltpu
from jax.experimental.pallas import tpu_sc as plsc

N = 10000
E = 320000
NF = 128
NH = 128
HEADS = 2
R = 16
ALPHA = 0.2
K = 3

NP = 10240
NC = 2
NS = 16
L = 16
C = 80
EPT = 20480
EPAD = NS * EPT
NCHUNK = EPT // C
C2 = C
NCH2 = NCHUNK
RPT = NP // NS
RCH = 64
NRC = RPT // RCH
AC = NH + L


_BR = 1024


def _dot(a, b, dims):
    return lax.dot_general(a, b, (dims, ((), ())),
                           preferred_element_type=jnp.float32)


def _prep_body(x_ref, rel_ref, w1_ref, b1_ref, w2_ref, b2_ref, cr_ref,
               wa_ref, wr_ref, aa_ref, wo_ref,
               t_ref, s3_ref, xw_ref, xrel_ref):
    xb = x_ref[...]
    rel = rel_ref[...]
    for h in range(HEADS):
        hid = jnp.maximum(_dot(rel, w1_ref[h], ((1,), (0,))) + b1_ref[h], 0.0)
        theta = _dot(hid, w2_ref[h], ((1,), (0,))) + b2_ref[h]
        ct = jnp.cos(theta)
        a = aa_ref[h]
        a1 = a[:, 0:NH]
        a2 = a[:, NH:2 * NH]
        a3 = a[:, 2 * NH:3 * NH]
        v1 = _dot(a1, wa_ref[h], ((1,), (1,)))
        v2 = _dot(a2, wr_ref[h], ((1,), (1,)))
        v3 = _dot(a3, wa_ref[h], ((1,), (1,)))
        s1 = _dot(xb, v1, ((1,), (1,)))
        s2 = _dot(xb, ct * v2, ((1,), (1,)))
        cterm = _dot(cr_ref[h], v2, ((1,), (1,)))
        t_ref[h] = s1 + s2 + cterm
        s3_ref[h] = _dot(xb, v3, ((1,), (1,)))
        xw_ref[h] = _dot(xb, wa_ref[h], ((1,), (0,)))
    xrel_ref[...] = _dot(rel, wo_ref[...], ((1,), (0,)))


def _prep(x_pad, rel, w1, b1, w2, b2, cr, wa, wr, aa, wo):
    full = lambda shape: pl.BlockSpec(shape, lambda i: tuple(0 for _ in shape))
    grid = NP // _BR
    return pl.pallas_call(
        _prep_body,
        grid=(grid,),
        in_specs=[
            pl.BlockSpec((_BR, NF), lambda i: (i, 0)),
            full((R, NF)),
            full((HEADS, NF, 512)),
            full((HEADS, 1, 512)),
            full((HEADS, 512, NF)),
            full((HEADS, 1, NF)),
            full((HEADS, 1, NF)),
            full((HEADS, NF, NH)),
            full((HEADS, NF, NH)),
            full((HEADS, 1, 3 * NH)),
            full((NF, HEADS * NH)),
        ],
        out_specs=[
            pl.BlockSpec((HEADS, _BR, R), lambda i: (0, i, 0)),
            pl.BlockSpec((HEADS, _BR, 1), lambda i: (0, i, 0)),
            pl.BlockSpec((HEADS, _BR, NH), lambda i: (0, i, 0)),
            pl.BlockSpec((R, HEADS * NH), lambda i: (0, 0)),
        ],
        out_shape=[
            jax.ShapeDtypeStruct((HEADS, NP, R), jnp.float32),
            jax.ShapeDtypeStruct((HEADS, NP, 1), jnp.float32),
            jax.ShapeDtypeStruct((HEADS, NP, NH), jnp.float32),
            jax.ShapeDtypeStruct((R, HEADS * NH), jnp.float32),
        ],
    )(x_pad, rel, w1, b1, w2, b2, cr, wa, wr, aa, wo)


_MESH = plsc.VectorSubcoreMesh(core_axis_name="c", subcore_axis_name="s")


def _zero_buf(buf, rows, cols):
    def zrow(i, carry):
        for j in range(cols // L):
            buf[i, pl.ds(j * L, L)] = jnp.zeros((L,), jnp.float32)
        return carry
    lax.fori_loop(0, rows, zrow, 0)


def _k1_body(src_h, dst_h, et_h, t_h, s3_h, xw_h,
             hl_h, hls_h, dis_h,
             acc_sh, rsum_sh, deg_sh,
             isrc4, idst4, iet4, iflat4, ixw4, tval2, s3val2, xra, xrb,
             e_a, e_b, ones_v, fin, disc, invc, rs_v, deg_v,
             si0, si1, si2, si3, st0, st1, sg0, sg1, ss0, ss1):
    c = lax.axis_index("c")
    s = lax.axis_index("s")
    semi = [si0, si1, si2, si3]
    semt = [st0, st1]
    semg = [sg0, sg1]
    sems = [ss0, ss1]
    xr = [xra, xrb]
    ev = [e_a, e_b]

    _zero_buf(fin, RCH, NH)
    for gg in range(RCH // L):
        rs_v[pl.ds(gg * L, L)] = jnp.zeros((L,), jnp.float32)
    for grp in range(C // L):
        ones_v[pl.ds(grp * L, L)] = jnp.ones((L,), jnp.float32)
    zds = []
    for k in range(NRC):
        rb = s * RPT + k * RCH
        zds.append(pltpu.async_copy(fin, acc_sh.at[pl.ds(rb, RCH)], si0))
        zds.append(pltpu.async_copy(rs_v, rsum_sh.at[pl.ds(rb, RCH)], st0))
        zds.append(pltpu.async_copy(rs_v, deg_sh.at[pl.ds(rb, RCH)], st1))
    for d in zds:
        d.wait()
    plsc.subcore_barrier()

    def issue_idx(g, j):
        base = s * EPT + g * C
        pltpu.async_copy(src_h.at[pl.ds(base, C)], isrc4.at[j], semi[j])
        pltpu.async_copy(dst_h.at[pl.ds(base, C)], idst4.at[j], semi[j])
        pltpu.async_copy(et_h.at[pl.ds(base, C)], iet4.at[j], semi[j])

    def wait_idx(j):
        pltpu.make_async_copy(src_h.at[pl.ds(0, C)], isrc4.at[j],
                              semi[j]).wait()
        pltpu.make_async_copy(dst_h.at[pl.ds(0, C)], idst4.at[j],
                              semi[j]).wait()
        pltpu.make_async_copy(et_h.at[pl.ds(0, C)], iet4.at[j],
                              semi[j]).wait()

    def comp_idx(j):
        for grp in range(C // L):
            sg = isrc4[j, pl.ds(grp * L, L)]
            etg = iet4[j, pl.ds(grp * L, L)]
            dg = idst4[j, pl.ds(grp * L, L)]
            iflat4[j, pl.ds(grp * L, L)] = (sg + c * NP) * R + etg
            ixw4[j, pl.ds(grp * L, L)] = dg + c * NP

    def issue_gathers(j, p):
        pltpu.async_copy(t_h.at[iflat4.at[j]], tval2.at[p], semt[p])
        pltpu.async_copy(s3_h.at[ixw4.at[j]], s3val2.at[p], semt[p])
        pltpu.async_copy(xw_h.at[ixw4.at[j]], xr[p], semg[p])

    def wait_gathers(p):
        pltpu.make_async_copy(t_h.at[pl.ds(0, C)], tval2.at[p],
                              semt[p]).wait()
        pltpu.make_async_copy(s3_h.at[pl.ds(0, C)], s3val2.at[p],
                              semt[p]).wait()
        pltpu.make_async_copy(xw_h.at[pl.ds(0, C)], xr[p], semg[p]).wait()

    def compute_e(p):
        for grp in range(C // L):
            tv = tval2[p, pl.ds(grp * L, L)]
            s3d = s3val2[p, pl.ds(grp * L, L)]
            logit = tv + s3d
            lr = jnp.where(logit >= 0.0, logit, ALPHA * logit)
            ev[p][pl.ds(grp * L, L)] = jnp.exp(-lr)

    def scale(p):
        xrp = xr[p]
        evp = ev[p]

        def scale_row(i, carry2):
            ei = evp[pl.ds(i, L)][0]
            for j in range(NH // L):
                xrp[i, pl.ds(j * L, L)] = xrp[i, pl.ds(j * L, L)] * ei
            return carry2
        lax.fori_loop(0, C, scale_row, 0)

    def issue_scatters(j, p):
        pltpu.async_copy(xr[p], acc_sh.at[isrc4.at[j]], sems[p], add=True)
        pltpu.async_copy(ev[p].at[pl.ds(0, C)], rsum_sh.at[isrc4.at[j]],
                         sems[p], add=True)
        pltpu.async_copy(ones_v, deg_sh.at[isrc4.at[j]], sems[p], add=True)

    def wait_scatters(p):
        pltpu.make_async_copy(xw_h.at[pl.ds(0, C)], xr[p], sems[p]).wait()
        pltpu.make_async_copy(t_h.at[pl.ds(0, C)], ev[p].at[pl.ds(0, C)],
                              sems[p]).wait()
        pltpu.make_async_copy(t_h.at[pl.ds(0, C)], ones_v, sems[p]).wait()

    def chunk_step(g, r, have_g1, have_g3, have_gm1):
        j, j1, j3 = r % 4, (r + 1) % 4, (r + 3) % 4
        p, p1 = r % 2, (r + 1) % 2
        if have_g1:
            wait_idx(j1)
            comp_idx(j1)
        if have_gm1:
            wait_scatters(p1)
        if have_g3:
            issue_idx(g + 3, j3)
        if have_g1:
            issue_gathers(j1, p1)
        wait_gathers(p)
        compute_e(p)
        scale(p)
        issue_scatters(j, p)

    issue_idx(0, 0)
    issue_idx(1, 1)
    issue_idx(2, 2)
    wait_idx(0)
    comp_idx(0)
    issue_gathers(0, 0)
    for r in range(4):
        chunk_step(r, r, True, True, r >= 1)

    def quad(q, carry):
        for r in range(4):
            chunk_step(q * 4 + r, r, True, True, True)
        return carry
    lax.fori_loop(1, NCHUNK // 4 - 1, quad, 0)

    for r in range(4):
        chunk_step(NCHUNK - 4 + r, r, r < 3, r < 1, True)
    wait_scatters(1)
    plsc.subcore_barrier()

    for k in range(NRC):
        rb = s * RPT + k * RCH
        d1 = pltpu.async_copy(acc_sh.at[pl.ds(rb, RCH)], fin, si0)
        d2 = pltpu.async_copy(rsum_sh.at[pl.ds(rb, RCH)], rs_v, st0)
        d3 = pltpu.async_copy(deg_sh.at[pl.ds(rb, RCH)], deg_v, st1)
        d1.wait()
        d2.wait()
        d3.wait()
        for gg in range(RCH // L):
            rs = rs_v[pl.ds(gg * L, L)]
            rs = jnp.where(rs == 0.0, 1e-12, rs)
            degv = deg_v[pl.ds(gg * L, L)] + 1.0
            ii = plsc.bitcast(degv, jnp.int32)
            ii = jnp.int32(0x5F3759DF) - lax.shift_right_arithmetic(ii, 1)
            y = plsc.bitcast(ii, jnp.float32)
            for _ in range(3):
                y = y * (1.5 - 0.5 * degv * y * y)
            invc[pl.ds(gg * L, L)] = 1.0 / rs
            disc[pl.ds(gg * L, L)] = y

        def fin_row(i, carry):
            iv = invc[pl.ds(i, L)][0]
            for j in range(NH // L):
                v = fin[i, pl.ds(j * L, L)] * iv
                fin[i, pl.ds(j * L, L)] = jnp.where(v > 0.0, v,
                                                    jnp.exp(v) - 1.0)
            return carry
        lax.fori_loop(0, RCH, fin_row, 0)
        pltpu.sync_copy(fin, hl_h.at[pl.ds(c * NP + rb, RCH)])

        def scl_row(i, carry):
            di = disc[pl.ds(i, L)][0]
            for j in range(NH // L):
                fin[i, pl.ds(j * L, L)] = fin[i, pl.ds(j * L, L)] * di
            return carry
        lax.fori_loop(0, RCH, scl_row, 0)
        pltpu.sync_copy(fin, hls_h.at[pl.ds(c * NP + rb, RCH)])

        @pl.when(c == 0)
        def _():
            pltpu.sync_copy(disc.at[pl.ds(0, RCH)], dis_h.at[pl.ds(rb, RCH)])


_k1 = functools.partial(
    pl.kernel,
    out_type=(jax.ShapeDtypeStruct((HEADS * NP, NH), jnp.float32),
              jax.ShapeDtypeStruct((HEADS * NP, NH), jnp.float32),
              jax.ShapeDtypeStruct((NP,), jnp.float32)),
    mesh=_MESH,
    compiler_params=pltpu.CompilerParams(needs_layout_passes=False),
    scratch_types=[
        pltpu.VMEM_SHARED((NP, NH), jnp.float32),
        pltpu.VMEM_SHARED((NP,), jnp.float32),
        pltpu.VMEM_SHARED((NP,), jnp.float32),
        pltpu.VMEM((4, C), jnp.int32),
        pltpu.VMEM((4, C), jnp.int32),
        pltpu.VMEM((4, C), jnp.int32),
        pltpu.VMEM((4, C), jnp.int32),
        pltpu.VMEM((4, C), jnp.int32),
        pltpu.VMEM((2, C), jnp.float32),
        pltpu.VMEM((2, C), jnp.float32),
        pltpu.VMEM((C, NH), jnp.float32),
        pltpu.VMEM((C, NH), jnp.float32),
        pltpu.VMEM((C + L,), jnp.float32),
        pltpu.VMEM((C + L,), jnp.float32),
        pltpu.VMEM((C,), jnp.float32),
        pltpu.VMEM((RCH, NH), jnp.float32),
        pltpu.VMEM((RCH + L,), jnp.float32),
        pltpu.VMEM((RCH + L,), jnp.float32),
        pltpu.VMEM((RCH,), jnp.float32),
        pltpu.VMEM((RCH,), jnp.float32),
        pltpu.SemaphoreType.DMA,
        pltpu.SemaphoreType.DMA,
        pltpu.SemaphoreType.DMA,
        pltpu.SemaphoreType.DMA,
        pltpu.SemaphoreType.DMA,
        pltpu.SemaphoreType.DMA,
        pltpu.SemaphoreType.DMA,
        pltpu.SemaphoreType.DMA,
        pltpu.SemaphoreType.DMA,
        pltpu.SemaphoreType.DMA,
    ],
)(_k1_body)


def _k2_body(src_h, dst_h, dis_h, hls_h,
             hg_h, a_h, b_h,
             acc_sh, disf, isrc4, idst4, ixw4, xra, xrb, fin, prev,
             si0, si1, si2, si3, sg0, sg1, ss0, ss1):
    c = lax.axis_index("c")
    s = lax.axis_index("s")
    semi = [si0, si1, si2, si3]
    semg = [sg0, sg1]
    sems = [ss0, ss1]
    xr = [xra, xrb]

    def issue_idx(g, j):
        base = s * EPT + g * C2
        pltpu.async_copy(src_h.at[pl.ds(base, C2)], isrc4.at[j], semi[j])
        pltpu.async_copy(dst_h.at[pl.ds(base, C2)], idst4.at[j], semi[j])

    def wait_idx(j):
        pltpu.make_async_copy(src_h.at[pl.ds(0, C2)], isrc4.at[j],
                              semi[j]).wait()
        pltpu.make_async_copy(dst_h.at[pl.ds(0, C2)], idst4.at[j],
                              semi[j]).wait()

    def comp_ixw(j):
        for grp in range(C2 // L):
            dg = idst4[j, pl.ds(grp * L, L)]
            ixw4[j, pl.ds(grp * L, L)] = dg + c * NP

    hops = [(hls_h, a_h), (a_h, b_h), (b_h, hg_h)]
    for hop, (src_ref, dst_ref) in enumerate(hops):
        _zero_buf(fin, RCH, NH)
        zds = []
        for k in range(NRC):
            zds.append(pltpu.async_copy(
                fin, acc_sh.at[pl.ds(s * RPT + k * RCH, RCH)], si0))
        for d in zds:
            d.wait()
        plsc.subcore_barrier()

        def issue_gather(j, p):
            pltpu.async_copy(src_ref.at[ixw4.at[j]], xr[p], semg[p])

        def wait_gather(p):
            pltpu.make_async_copy(src_ref.at[pl.ds(0, C2)], xr[p],
                                  semg[p]).wait()

        def issue_scatter(j, p):
            pltpu.async_copy(xr[p], acc_sh.at[isrc4.at[j]], sems[p], add=True)

        def wait_scatter(p):
            pltpu.make_async_copy(src_ref.at[pl.ds(0, C2)], xr[p],
                                  sems[p]).wait()

        def chunk_step(g, r, have_g1, have_g3, have_gm1):
            j, j1, j3 = r % 4, (r + 1) % 4, (r + 3) % 4
            p, p1 = r % 2, (r + 1) % 2
            if have_g1:
                wait_idx(j1)
                comp_ixw(j1)
            if have_gm1:
                wait_scatter(p1)
            if have_g3:
                issue_idx(g + 3, j3)
            if have_g1:
                issue_gather(j1, p1)
            wait_gather(p)
            issue_scatter(j, p)

        issue_idx(0, 0)
        issue_idx(1, 1)
        issue_idx(2, 2)
        wait_idx(0)
        comp_ixw(0)
        issue_gather(0, 0)
        for r in range(4):
            chunk_step(r, r, True, True, r >= 1)

        def quad(q, carry):
            for r in range(4):
                chunk_step(q * 4 + r, r, True, True, True)
            return carry
        lax.fori_loop(1, NCH2 // 4 - 1, quad, 0)

        for r in range(4):
            g = NCH2 - 4 + r
            chunk_step(g, r, r < 3, r < 1, True)
        wait_scatter(1)
        plsc.subcore_barrier()

        for k in range(NRC):
            rb = s * RPT + k * RCH
            d1 = pltpu.async_copy(acc_sh.at[pl.ds(rb, RCH)], fin, si0)
            d2 = pltpu.async_copy(src_ref.at[pl.ds(c * NP + rb, RCH)], prev,
                                  si1)
            d3 = pltpu.async_copy(dis_h.at[pl.ds(rb, RCH)],
                                  disf.at[pl.ds(0, RCH)], si2)
            d1.wait()
            d2.wait()
            d3.wait()

            def fin_row(i, carry):
                di = disf[pl.ds(i, L)][0]
                sw = di * di if hop < K - 1 else di
                for j in range(NH // L):
                    fin[i, pl.ds(j * L, L)] = (fin[i, pl.ds(j * L, L)] +
                                               prev[i, pl.ds(j * L, L)]) * sw
                return carry
            lax.fori_loop(0, RCH, fin_row, 0)
            pltpu.sync_copy(fin, dst_ref.at[pl.ds(c * NP + rb, RCH)])
        plsc.subcore_barrier()


_k2 = functools.partial(
    pl.kernel,
    out_type=(jax.ShapeDtypeStruct((HEADS * NP, NH), jnp.float32),
              jax.ShapeDtypeStruct((HEADS * NP, NH), jnp.float32),
              jax.ShapeDtypeStruct((HEADS * NP, NH), jnp.float32)),
    mesh=_MESH,
    compiler_params=pltpu.CompilerParams(needs_layout_passes=False),
    scratch_types=[
        pltpu.VMEM_SHARED((NP, NH), jnp.float32),
        pltpu.VMEM((RCH + L,), jnp.float32),
        pltpu.VMEM((4, C2), jnp.int32),
        pltpu.VMEM((4, C2), jnp.int32),
        pltpu.VMEM((4, C2), jnp.int32),
        pltpu.VMEM((C2, NH), jnp.float32),
        pltpu.VMEM((C2, NH), jnp.float32),
        pltpu.VMEM((RCH, NH), jnp.float32),
        pltpu.VMEM((RCH, NH), jnp.float32),
        pltpu.SemaphoreType.DMA,
        pltpu.SemaphoreType.DMA,
        pltpu.SemaphoreType.DMA,
        pltpu.SemaphoreType.DMA,
        pltpu.SemaphoreType.DMA,
        pltpu.SemaphoreType.DMA,
        pltpu.SemaphoreType.DMA,
        pltpu.SemaphoreType.DMA,
    ],
)(_k2_body)


def _fuse_body(hl_ref, hg_ref, w_ref, b_ref, out_ref):
    acc = b_ref[...]
    acc = acc + _dot(hl_ref[0], w_ref[0:NH, :], ((1,), (0,)))
    acc = acc + _dot(hl_ref[1], w_ref[NH:2 * NH, :], ((1,), (0,)))
    acc = acc + _dot(hg_ref[0], w_ref[2 * NH:3 * NH, :], ((1,), (0,)))
    acc = acc + _dot(hg_ref[1], w_ref[3 * NH:4 * NH, :], ((1,), (0,)))
    out_ref[...] = jnp.where(acc > 0.0, acc, jnp.exp(acc) - 1.0)


def _fuse(hl, hg, fw, fb):
    grid = NP // _BR
    return pl.pallas_call(
        _fuse_body,
        grid=(grid,),
        in_specs=[
            pl.BlockSpec((HEADS, _BR, NH), lambda i: (0, i, 0)),
            pl.BlockSpec((HEADS, _BR, NH), lambda i: (0, i, 0)),
            pl.BlockSpec((2 * HEADS * NH, HEADS * NH), lambda i: (0, 0)),
            pl.BlockSpec((1, HEADS * NH), lambda i: (0, 0)),
        ],
        out_specs=pl.BlockSpec((_BR, HEADS * NH), lambda i: (i, 0)),
        out_shape=jax.ShapeDtypeStruct((NP, HEADS * NH), jnp.float32),
    )(hl, hg, fw, fb)


@jax.jit
def kernel(entity_embeddings, relation_embeddings, edge_list, edge_type,
           thW1, thb1, thW2, thb2, c_r, W_att, W_r, a_att, W_out,
           fuse_W, fuse_b):
    pad = N + (jnp.arange(EPAD - E, dtype=jnp.int32) % (NP - N))
    src = jnp.concatenate([edge_list[0], pad])
    dst = jnp.concatenate([edge_list[1], pad])
    et = jnp.concatenate([edge_type, jnp.zeros((EPAD - E,), jnp.int32)])
    x_pad = jnp.pad(entity_embeddings, ((0, NP - N), (0, 0)))
    t_tab, s3_tab, xw_tab, x_rel = _prep(
        x_pad, relation_embeddings,
        thW1, thb1.reshape(HEADS, 1, 512), thW2, thb2.reshape(HEADS, 1, NF),
        c_r, W_att, W_r, a_att, W_out)
    s3_tab = s3_tab.reshape(HEADS * NP)
    t_tab = t_tab.reshape(HEADS * NP * R)
    xw_tab = xw_tab.reshape(HEADS * NP, NH)
    hl, hls, dis = _k1(src, dst, et, t_tab, s3_tab, xw_tab)
    hg, _, _ = _k2(src, dst, dis, hls)
    h_fused = _fuse(hl.reshape(HEADS, NP, NH), hg.reshape(HEADS, NP, NH),
                    fuse_W, fuse_b.reshape(1, HEADS * NH))
    return (h_fused[:N], x_rel)


# speedup vs baseline: 3.5471x; 1.0440x over previous
import functools

import jax
import jax.numpy as jnp
from jax import lax
from jax.experimental import pallas as pl
from jax.experimental.pallas import tpu as pltpu
from jax.experimental.pallas import tpu_sc as plsc

N = 10000
E = 320000
NF = 128
NH = 128
HEADS = 2
R = 16
ALPHA = 0.2
K = 3

NP = 10240
NC = 2
NS = 16
L = 16
C = 128
EPT = 20480
EPAD = NS * EPT
NCHUNK = EPT // C
C2 = C
NCH2 = NCHUNK
RPT = NP // NS
RCH = 32
NRC = RPT // RCH
AC = NH + L


_BR = 1024


def _dot(a, b, dims):
    return lax.dot_general(a, b, (dims, ((), ())),
                           preferred_element_type=jnp.float32)


def _prep_body(x_ref, rel_ref, w1_ref, b1_ref, w2_ref, b2_ref, cr_ref,
               wa_ref, wr_ref, aa_ref, wo_ref,
               t_ref, s3_ref, xw_ref, xrel_ref):
    xb = x_ref[...]
    rel = rel_ref[...]
    for h in range(HEADS):
        hid = jnp.maximum(_dot(rel, w1_ref[h], ((1,), (0,))) + b1_ref[h], 0.0)
        theta = _dot(hid, w2_ref[h], ((1,), (0,))) + b2_ref[h]
        ct = jnp.cos(theta)
        a = aa_ref[h]
        a1 = a[:, 0:NH]
        a2 = a[:, NH:2 * NH]
        a3 = a[:, 2 * NH:3 * NH]
        v1 = _dot(a1, wa_ref[h], ((1,), (1,)))
        v2 = _dot(a2, wr_ref[h], ((1,), (1,)))
        v3 = _dot(a3, wa_ref[h], ((1,), (1,)))
        s1 = _dot(xb, v1, ((1,), (1,)))
        s2 = _dot(xb, ct * v2, ((1,), (1,)))
        cterm = _dot(cr_ref[h], v2, ((1,), (1,)))
        t_ref[h] = s1 + s2 + cterm
        s3_ref[h] = _dot(xb, v3, ((1,), (1,)))
        xw_ref[h] = _dot(xb, wa_ref[h], ((1,), (0,)))
    xrel_ref[...] = _dot(rel, wo_ref[...], ((1,), (0,)))


def _prep(x_pad, rel, w1, b1, w2, b2, cr, wa, wr, aa, wo):
    full = lambda shape: pl.BlockSpec(shape, lambda i: tuple(0 for _ in shape))
    grid = NP // _BR
    return pl.pallas_call(
        _prep_body,
        grid=(grid,),
        in_specs=[
            pl.BlockSpec((_BR, NF), lambda i: (i, 0)),
            full((R, NF)),
            full((HEADS, NF, 512)),
            full((HEADS, 1, 512)),
            full((HEADS, 512, NF)),
            full((HEADS, 1, NF)),
            full((HEADS, 1, NF)),
            full((HEADS, NF, NH)),
            full((HEADS, NF, NH)),
            full((HEADS, 1, 3 * NH)),
            full((NF, HEADS * NH)),
        ],
        out_specs=[
            pl.BlockSpec((HEADS, _BR, R), lambda i: (0, i, 0)),
            pl.BlockSpec((HEADS, _BR, 1), lambda i: (0, i, 0)),
            pl.BlockSpec((HEADS, _BR, NH), lambda i: (0, i, 0)),
            pl.BlockSpec((R, HEADS * NH), lambda i: (0, 0)),
        ],
        out_shape=[
            jax.ShapeDtypeStruct((HEADS, NP, R), jnp.float32),
            jax.ShapeDtypeStruct((HEADS, NP, 1), jnp.float32),
            jax.ShapeDtypeStruct((HEADS, NP, NH), jnp.float32),
            jax.ShapeDtypeStruct((R, HEADS * NH), jnp.float32),
        ],
    )(x_pad, rel, w1, b1, w2, b2, cr, wa, wr, aa, wo)


_MESH = plsc.VectorSubcoreMesh(core_axis_name="c", subcore_axis_name="s")


def _zero_buf(buf, rows, cols):
    def zrow(i, carry):
        for j in range(cols // L):
            buf[i, pl.ds(j * L, L)] = jnp.zeros((L,), jnp.float32)
        return carry
    lax.fori_loop(0, rows, zrow, 0)


def _k1_body(src_h, dst_h, et_h, t_h, s3_h, xw_h,
             hl_h, hls_h, dis_h,
             acc_sh, rsum_sh, deg_sh,
             isrc4, idst4, iet4, iflat4, ixw4, tval2, s3val2, xra, xrb,
             e_a, e_b, ones_v, fin, disc, invc, rs_v, deg_v,
             si0, si1, si2, si3, st0, st1, sg0, sg1, ss0, ss1):
    c = lax.axis_index("c")
    s = lax.axis_index("s")
    semi = [si0, si1, si2, si3]
    semt = [st0, st1]
    semg = [sg0, sg1]
    sems = [ss0, ss1]
    xr = [xra, xrb]
    ev = [e_a, e_b]

    _zero_buf(fin, RCH, NH)
    for gg in range(RCH // L):
        rs_v[pl.ds(gg * L, L)] = jnp.zeros((L,), jnp.float32)
    for grp in range(C // L):
        ones_v[pl.ds(grp * L, L)] = jnp.ones((L,), jnp.float32)
    zds = []
    for k in range(NRC):
        rb = s * RPT + k * RCH
        zds.append(pltpu.async_copy(fin, acc_sh.at[pl.ds(rb, RCH)], si0))
        zds.append(pltpu.async_copy(rs_v, rsum_sh.at[pl.ds(rb, RCH)], st0))
        zds.append(pltpu.async_copy(rs_v, deg_sh.at[pl.ds(rb, RCH)], st1))
    for d in zds:
        d.wait()
    plsc.subcore_barrier()

    def issue_idx(g, j):
        base = s * EPT + g * C
        pltpu.async_copy(src_h.at[pl.ds(base, C)], isrc4.at[j], semi[j])
        pltpu.async_copy(dst_h.at[pl.ds(base, C)], idst4.at[j], semi[j])
        pltpu.async_copy(et_h.at[pl.ds(base, C)], iet4.at[j], semi[j])

    def wait_idx(j):
        pltpu.make_async_copy(src_h.at[pl.ds(0, C)], isrc4.at[j],
                              semi[j]).wait()
        pltpu.make_async_copy(dst_h.at[pl.ds(0, C)], idst4.at[j],
                              semi[j]).wait()
        pltpu.make_async_copy(et_h.at[pl.ds(0, C)], iet4.at[j],
                              semi[j]).wait()

    def comp_idx(j):
        for grp in range(C // L):
            sg = isrc4[j, pl.ds(grp * L, L)]
            etg = iet4[j, pl.ds(grp * L, L)]
            dg = idst4[j, pl.ds(grp * L, L)]
            iflat4[j, pl.ds(grp * L, L)] = (sg + c * NP) * R + etg
            ixw4[j, pl.ds(grp * L, L)] = dg + c * NP

    def issue_gathers(j, p):
        pltpu.async_copy(t_h.at[iflat4.at[j]], tval2.at[p], semt[p])
        pltpu.async_copy(s3_h.at[ixw4.at[j]], s3val2.at[p], semt[p])
        pltpu.async_copy(xw_h.at[ixw4.at[j]], xr[p], semg[p])

    def wait_gathers(p):
        pltpu.make_async_copy(t_h.at[pl.ds(0, C)], tval2.at[p],
                              semt[p]).wait()
        pltpu.make_async_copy(s3_h.at[pl.ds(0, C)], s3val2.at[p],
                              semt[p]).wait()
        pltpu.make_async_copy(xw_h.at[pl.ds(0, C)], xr[p], semg[p]).wait()

    def compute_e(p):
        for grp in range(C // L):
            tv = tval2[p, pl.ds(grp * L, L)]
            s3d = s3val2[p, pl.ds(grp * L, L)]
            logit = tv + s3d
            lr = jnp.where(logit >= 0.0, logit, ALPHA * logit)
            ev[p][pl.ds(grp * L, L)] = jnp.exp(-lr)

    def scale(p):
        xrp = xr[p]
        evp = ev[p]

        def scale_row(i, carry2):
            ei = evp[pl.ds(i, L)][0]
            for j in range(NH // L):
                xrp[i, pl.ds(j * L, L)] = xrp[i, pl.ds(j * L, L)] * ei
            return carry2
        lax.fori_loop(0, C, scale_row, 0)

    def issue_scatters(j, p):
        pltpu.async_copy(xr[p], acc_sh.at[isrc4.at[j]], sems[p], add=True)
        pltpu.async_copy(ev[p].at[pl.ds(0, C)], rsum_sh.at[isrc4.at[j]],
                         sems[p], add=True)
        pltpu.async_copy(ones_v, deg_sh.at[isrc4.at[j]], sems[p], add=True)

    def wait_scatters(p):
        pltpu.make_async_copy(xw_h.at[pl.ds(0, C)], xr[p], sems[p]).wait()
        pltpu.make_async_copy(t_h.at[pl.ds(0, C)], ev[p].at[pl.ds(0, C)],
                              sems[p]).wait()
        pltpu.make_async_copy(t_h.at[pl.ds(0, C)], ones_v, sems[p]).wait()

    def chunk_step(g, r, have_g1, have_g3, have_gm1):
        j, j1, j3 = r % 4, (r + 1) % 4, (r + 3) % 4
        p, p1 = r % 2, (r + 1) % 2
        if have_g1:
            wait_idx(j1)
            comp_idx(j1)
        if have_gm1:
            wait_scatters(p1)
        if have_g3:
            issue_idx(g + 3, j3)
        if have_g1:
            issue_gathers(j1, p1)
        wait_gathers(p)
        compute_e(p)
        scale(p)
        issue_scatters(j, p)

    issue_idx(0, 0)
    issue_idx(1, 1)
    issue_idx(2, 2)
    wait_idx(0)
    comp_idx(0)
    issue_gathers(0, 0)
    for r in range(4):
        chunk_step(r, r, True, True, r >= 1)

    def quad(q, carry):
        for r in range(4):
            chunk_step(q * 4 + r, r, True, True, True)
        return carry
    lax.fori_loop(1, NCHUNK // 4 - 1, quad, 0)

    for r in range(4):
        chunk_step(NCHUNK - 4 + r, r, r < 3, r < 1, True)
    wait_scatters(1)
    plsc.subcore_barrier()

    for k in range(NRC):
        rb = s * RPT + k * RCH
        d1 = pltpu.async_copy(acc_sh.at[pl.ds(rb, RCH)], fin, si0)
        d2 = pltpu.async_copy(rsum_sh.at[pl.ds(rb, RCH)], rs_v, st0)
        d3 = pltpu.async_copy(deg_sh.at[pl.ds(rb, RCH)], deg_v, st1)
        d1.wait()
        d2.wait()
        d3.wait()
        for gg in range(RCH // L):
            rs = rs_v[pl.ds(gg * L, L)]
            rs = jnp.where(rs == 0.0, 1e-12, rs)
            degv = deg_v[pl.ds(gg * L, L)] + 1.0
            ii = plsc.bitcast(degv, jnp.int32)
            ii = jnp.int32(0x5F3759DF) - lax.shift_right_arithmetic(ii, 1)
            y = plsc.bitcast(ii, jnp.float32)
            for _ in range(3):
                y = y * (1.5 - 0.5 * degv * y * y)
            invc[pl.ds(gg * L, L)] = 1.0 / rs
            disc[pl.ds(gg * L, L)] = y

        def fin_row(i, carry):
            iv = invc[pl.ds(i, L)][0]
            for j in range(NH // L):
                v = fin[i, pl.ds(j * L, L)] * iv
                fin[i, pl.ds(j * L, L)] = jnp.where(v > 0.0, v,
                                                    jnp.exp(v) - 1.0)
            return carry
        lax.fori_loop(0, RCH, fin_row, 0)
        pltpu.sync_copy(fin, hl_h.at[pl.ds(c * NP + rb, RCH)])

        def scl_row(i, carry):
            di = disc[pl.ds(i, L)][0]
            for j in range(NH // L):
                fin[i, pl.ds(j * L, L)] = fin[i, pl.ds(j * L, L)] * di
            return carry
        lax.fori_loop(0, RCH, scl_row, 0)
        pltpu.sync_copy(fin, hls_h.at[pl.ds(c * NP + rb, RCH)])

        @pl.when(c == 0)
        def _():
            pltpu.sync_copy(disc.at[pl.ds(0, RCH)], dis_h.at[pl.ds(rb, RCH)])


_k1 = functools.partial(
    pl.kernel,
    out_type=(jax.ShapeDtypeStruct((HEADS * NP, NH), jnp.float32),
              jax.ShapeDtypeStruct((HEADS * NP, NH), jnp.float32),
              jax.ShapeDtypeStruct((NP,), jnp.float32)),
    mesh=_MESH,
    compiler_params=pltpu.CompilerParams(needs_layout_passes=False),
    scratch_types=[
        pltpu.VMEM_SHARED((NP, NH), jnp.float32),
        pltpu.VMEM_SHARED((NP,), jnp.float32),
        pltpu.VMEM_SHARED((NP,), jnp.float32),
        pltpu.VMEM((4, C), jnp.int32),
        pltpu.VMEM((4, C), jnp.int32),
        pltpu.VMEM((4, C), jnp.int32),
        pltpu.VMEM((4, C), jnp.int32),
        pltpu.VMEM((4, C), jnp.int32),
        pltpu.VMEM((2, C), jnp.float32),
        pltpu.VMEM((2, C), jnp.float32),
        pltpu.VMEM((C, NH), jnp.float32),
        pltpu.VMEM((C, NH), jnp.float32),
        pltpu.VMEM((C + L,), jnp.float32),
        pltpu.VMEM((C + L,), jnp.float32),
        pltpu.VMEM((C,), jnp.float32),
        pltpu.VMEM((RCH, NH), jnp.float32),
        pltpu.VMEM((RCH + L,), jnp.float32),
        pltpu.VMEM((RCH + L,), jnp.float32),
        pltpu.VMEM((RCH,), jnp.float32),
        pltpu.VMEM((RCH,), jnp.float32),
        pltpu.SemaphoreType.DMA,
        pltpu.SemaphoreType.DMA,
        pltpu.SemaphoreType.DMA,
        pltpu.SemaphoreType.DMA,
        pltpu.SemaphoreType.DMA,
        pltpu.SemaphoreType.DMA,
        pltpu.SemaphoreType.DMA,
        pltpu.SemaphoreType.DMA,
        pltpu.SemaphoreType.DMA,
        pltpu.SemaphoreType.DMA,
    ],
)(_k1_body)


def _k2_body(src_h, dst_h, dis_h, hls_h,
             hg_h, a_h, b_h,
             acc_sh, disf, isrc4, idst4, ixw4, xra, xrb, fin, prev,
             si0, si1, si2, si3, sg0, sg1, ss0, ss1):
    c = lax.axis_index("c")
    s = lax.axis_index("s")
    semi = [si0, si1, si2, si3]
    semg = [sg0, sg1]
    sems = [ss0, ss1]
    xr = [xra, xrb]

    def issue_idx(g, j):
        base = s * EPT + g * C2
        pltpu.async_copy(src_h.at[pl.ds(base, C2)], isrc4.at[j], semi[j])
        pltpu.async_copy(dst_h.at[pl.ds(base, C2)], idst4.at[j], semi[j])

    def wait_idx(j):
        pltpu.make_async_copy(src_h.at[pl.ds(0, C2)], isrc4.at[j],
                              semi[j]).wait()
        pltpu.make_async_copy(dst_h.at[pl.ds(0, C2)], idst4.at[j],
                              semi[j]).wait()

    def comp_ixw(j):
        for grp in range(C2 // L):
            dg = idst4[j, pl.ds(grp * L, L)]
            ixw4[j, pl.ds(grp * L, L)] = dg + c * NP

    hops = [(hls_h, a_h), (a_h, b_h), (b_h, hg_h)]
    for hop, (src_ref, dst_ref) in enumerate(hops):
        _zero_buf(fin, RCH, NH)
        zds = []
        for k in range(NRC):
            zds.append(pltpu.async_copy(
                fin, acc_sh.at[pl.ds(s * RPT + k * RCH, RCH)], si0))
        for d in zds:
            d.wait()
        plsc.subcore_barrier()

        def issue_gather(j, p):
            pltpu.async_copy(src_ref.at[ixw4.at[j]], xr[p], semg[p])

        def wait_gather(p):
            pltpu.make_async_copy(src_ref.at[pl.ds(0, C2)], xr[p],
                                  semg[p]).wait()

        def issue_scatter(j, p):
            pltpu.async_copy(xr[p], acc_sh.at[isrc4.at[j]], sems[p], add=True)

        def wait_scatter(p):
            pltpu.make_async_copy(src_ref.at[pl.ds(0, C2)], xr[p],
                                  sems[p]).wait()

        def chunk_step(g, r, have_g1, have_g3, have_gm1):
            j, j1, j3 = r % 4, (r + 1) % 4, (r + 3) % 4
            p, p1 = r % 2, (r + 1) % 2
            if have_g1:
                wait_idx(j1)
                comp_ixw(j1)
            if have_gm1:
                wait_scatter(p1)
            if have_g3:
                issue_idx(g + 3, j3)
            if have_g1:
                issue_gather(j1, p1)
            wait_gather(p)
            issue_scatter(j, p)

        issue_idx(0, 0)
        issue_idx(1, 1)
        issue_idx(2, 2)
        wait_idx(0)
        comp_ixw(0)
        issue_gather(0, 0)
        for r in range(4):
            chunk_step(r, r, True, True, r >= 1)

        def quad(q, carry):
            for r in range(4):
                chunk_step(q * 4 + r, r, True, True, True)
            return carry
        lax.fori_loop(1, NCH2 // 4 - 1, quad, 0)

        for r in range(4):
            g = NCH2 - 4 + r
            chunk_step(g, r, r < 3, r < 1, True)
        wait_scatter(1)
        plsc.subcore_barrier()

        for k in range(NRC):
            rb = s * RPT + k * RCH
            d1 = pltpu.async_copy(acc_sh.at[pl.ds(rb, RCH)], fin, si0)
            d2 = pltpu.async_copy(src_ref.at[pl.ds(c * NP + rb, RCH)], prev,
                                  si1)
            d3 = pltpu.async_copy(dis_h.at[pl.ds(rb, RCH)],
                                  disf.at[pl.ds(0, RCH)], si2)
            d1.wait()
            d2.wait()
            d3.wait()

            def fin_row(i, carry):
                di = disf[pl.ds(i, L)][0]
                sw = di * di if hop < K - 1 else di
                for j in range(NH // L):
                    fin[i, pl.ds(j * L, L)] = (fin[i, pl.ds(j * L, L)] +
                                               prev[i, pl.ds(j * L, L)]) * sw
                return carry
            lax.fori_loop(0, RCH, fin_row, 0)
            pltpu.sync_copy(fin, dst_ref.at[pl.ds(c * NP + rb, RCH)])
        plsc.subcore_barrier()


_k2 = functools.partial(
    pl.kernel,
    out_type=(jax.ShapeDtypeStruct((HEADS * NP, NH), jnp.float32),
              jax.ShapeDtypeStruct((HEADS * NP, NH), jnp.float32),
              jax.ShapeDtypeStruct((HEADS * NP, NH), jnp.float32)),
    mesh=_MESH,
    compiler_params=pltpu.CompilerParams(needs_layout_passes=False),
    scratch_types=[
        pltpu.VMEM_SHARED((NP, NH), jnp.float32),
        pltpu.VMEM((RCH + L,), jnp.float32),
        pltpu.VMEM((4, C2), jnp.int32),
        pltpu.VMEM((4, C2), jnp.int32),
        pltpu.VMEM((4, C2), jnp.int32),
        pltpu.VMEM((C2, NH), jnp.float32),
        pltpu.VMEM((C2, NH), jnp.float32),
        pltpu.VMEM((RCH, NH), jnp.float32),
        pltpu.VMEM((RCH, NH), jnp.float32),
        pltpu.SemaphoreType.DMA,
        pltpu.SemaphoreType.DMA,
        pltpu.SemaphoreType.DMA,
        pltpu.SemaphoreType.DMA,
        pltpu.SemaphoreType.DMA,
        pltpu.SemaphoreType.DMA,
        pltpu.SemaphoreType.DMA,
        pltpu.SemaphoreType.DMA,
    ],
)(_k2_body)


def _fuse_body(hl_ref, hg_ref, w_ref, b_ref, out_ref):
    acc = b_ref[...]
    acc = acc + _dot(hl_ref[0], w_ref[0:NH, :], ((1,), (0,)))
    acc = acc + _dot(hl_ref[1], w_ref[NH:2 * NH, :], ((1,), (0,)))
    acc = acc + _dot(hg_ref[0], w_ref[2 * NH:3 * NH, :], ((1,), (0,)))
    acc = acc + _dot(hg_ref[1], w_ref[3 * NH:4 * NH, :], ((1,), (0,)))
    out_ref[...] = jnp.where(acc > 0.0, acc, jnp.exp(acc) - 1.0)


def _fuse(hl, hg, fw, fb):
    grid = NP // _BR
    return pl.pallas_call(
        _fuse_body,
        grid=(grid,),
        in_specs=[
            pl.BlockSpec((HEADS, _BR, NH), lambda i: (0, i, 0)),
            pl.BlockSpec((HEADS, _BR, NH), lambda i: (0, i, 0)),
            pl.BlockSpec((2 * HEADS * NH, HEADS * NH), lambda i: (0, 0)),
            pl.BlockSpec((1, HEADS * NH), lambda i: (0, 0)),
        ],
        out_specs=pl.BlockSpec((_BR, HEADS * NH), lambda i: (i, 0)),
        out_shape=jax.ShapeDtypeStruct((NP, HEADS * NH), jnp.float32),
    )(hl, hg, fw, fb)


@jax.jit
def kernel(entity_embeddings, relation_embeddings, edge_list, edge_type,
           thW1, thb1, thW2, thb2, c_r, W_att, W_r, a_att, W_out,
           fuse_W, fuse_b):
    pad = N + (jnp.arange(EPAD - E, dtype=jnp.int32) % (NP - N))
    src = jnp.concatenate([edge_list[0], pad])
    dst = jnp.concatenate([edge_list[1], pad])
    et = jnp.concatenate([edge_type, jnp.zeros((EPAD - E,), jnp.int32)])
    x_pad = jnp.pad(entity_embeddings, ((0, NP - N), (0, 0)))
    t_tab, s3_tab, xw_tab, x_rel = _prep(
        x_pad, relation_embeddings,
        thW1, thb1.reshape(HEADS, 1, 512), thW2, thb2.reshape(HEADS, 1, NF),
        c_r, W_att, W_r, a_att, W_out)
    s3_tab = s3_tab.reshape(HEADS * NP)
    t_tab = t_tab.reshape(HEADS * NP * R)
    xw_tab = xw_tab.reshape(HEADS * NP, NH)
    hl, hls, dis = _k1(src, dst, et, t_tab, s3_tab, xw_tab)
    hg, _, _ = _k2(src, dst, dis, hls)
    h_fused = _fuse(hl.reshape(HEADS, NP, NH), hg.reshape(HEADS, NP, NH),
                    fuse_W, fuse_b.reshape(1, HEADS * NH))
    return (h_fused[:N], x_rel)


# speedup vs baseline: 3.9037x; 1.1005x over previous
import functools

import jax
import jax.numpy as jnp
from jax import lax
from jax.experimental import pallas as pl
from jax.experimental.pallas import tpu as pltpu
from jax.experimental.pallas import tpu_sc as plsc

N = 10000
E = 320000
NF = 128
NH = 128
HEADS = 2
R = 16
ALPHA = 0.2
K = 3

NP = 10240
NC = 2
NS = 16
L = 16
C = 128
EPT = 20480
EPAD = NS * EPT
NCHUNK = EPT // C
C2 = C
NCH2 = NCHUNK
RPT = NP // NS
RCH = 32
NRC = RPT // RCH
AC = NH + L


_BR = 1024


def _dot(a, b, dims):
    return lax.dot_general(a, b, (dims, ((), ())),
                           preferred_element_type=jnp.float32)


def _prep_body(x_ref, rel_ref, w1_ref, b1_ref, w2_ref, b2_ref, cr_ref,
               wa_ref, wr_ref, aa_ref, wo_ref,
               t_ref, s3_ref, xw_ref, xrel_ref):
    xb = x_ref[...]
    rel = rel_ref[...]
    for h in range(HEADS):
        hid = jnp.maximum(_dot(rel, w1_ref[h], ((1,), (0,))) + b1_ref[h], 0.0)
        theta = _dot(hid, w2_ref[h], ((1,), (0,))) + b2_ref[h]
        ct = jnp.cos(theta)
        a = aa_ref[h]
        a1 = a[:, 0:NH]
        a2 = a[:, NH:2 * NH]
        a3 = a[:, 2 * NH:3 * NH]
        v1 = _dot(a1, wa_ref[h], ((1,), (1,)))
        v2 = _dot(a2, wr_ref[h], ((1,), (1,)))
        v3 = _dot(a3, wa_ref[h], ((1,), (1,)))
        s1 = _dot(xb, v1, ((1,), (1,)))
        s2 = _dot(xb, ct * v2, ((1,), (1,)))
        cterm = _dot(cr_ref[h], v2, ((1,), (1,)))
        t_ref[h] = s1 + s2 + cterm
        s3_ref[h] = _dot(xb, v3, ((1,), (1,)))
        xw_ref[h] = _dot(xb, wa_ref[h], ((1,), (0,)))
    xrel_ref[...] = _dot(rel, wo_ref[...], ((1,), (0,)))


def _prep(x_pad, rel, w1, b1, w2, b2, cr, wa, wr, aa, wo):
    full = lambda shape: pl.BlockSpec(shape, lambda i: tuple(0 for _ in shape))
    grid = NP // _BR
    return pl.pallas_call(
        _prep_body,
        grid=(grid,),
        in_specs=[
            pl.BlockSpec((_BR, NF), lambda i: (i, 0)),
            full((R, NF)),
            full((HEADS, NF, 512)),
            full((HEADS, 1, 512)),
            full((HEADS, 512, NF)),
            full((HEADS, 1, NF)),
            full((HEADS, 1, NF)),
            full((HEADS, NF, NH)),
            full((HEADS, NF, NH)),
            full((HEADS, 1, 3 * NH)),
            full((NF, HEADS * NH)),
        ],
        out_specs=[
            pl.BlockSpec((HEADS, _BR, R), lambda i: (0, i, 0)),
            pl.BlockSpec((HEADS, _BR, 1), lambda i: (0, i, 0)),
            pl.BlockSpec((HEADS, _BR, NH), lambda i: (0, i, 0)),
            pl.BlockSpec((R, HEADS * NH), lambda i: (0, 0)),
        ],
        out_shape=[
            jax.ShapeDtypeStruct((HEADS, NP, R), jnp.float32),
            jax.ShapeDtypeStruct((HEADS, NP, 1), jnp.float32),
            jax.ShapeDtypeStruct((HEADS, NP, NH), jnp.float32),
            jax.ShapeDtypeStruct((R, HEADS * NH), jnp.float32),
        ],
    )(x_pad, rel, w1, b1, w2, b2, cr, wa, wr, aa, wo)


_MESH = plsc.VectorSubcoreMesh(core_axis_name="c", subcore_axis_name="s")


def _zero_buf(buf, rows, cols):
    def zrow(i, carry):
        for j in range(cols // L):
            buf[i, pl.ds(j * L, L)] = jnp.zeros((L,), jnp.float32)
        return carry
    lax.fori_loop(0, rows, zrow, 0)


def _k1_body(src_h, dst_h, et_h, t_h, s3_h, xw_h,
             hl_h, hls_h, dis_h,
             acc_sh, rsum_sh, deg_sh,
             isrc4, idst4, iet4, iflat4, ixw4, tval2, s3val2, xra, xrb,
             e_a, e_b, ones_v, fin, disc, invc, rs_v, deg_v,
             si0, si1, si2, si3, st0, st1, sg0, sg1, ss0, ss1):
    c = lax.axis_index("c")
    s = lax.axis_index("s")
    semi = [si0, si1, si2, si3]
    semt = [st0, st1]
    semg = [sg0, sg1]
    sems = [ss0, ss1]
    xr = [xra, xrb]
    ev = [e_a, e_b]

    _zero_buf(fin, RCH, NH)
    for gg in range(RCH // L):
        rs_v[pl.ds(gg * L, L)] = jnp.zeros((L,), jnp.float32)
    for grp in range(C // L):
        ones_v[pl.ds(grp * L, L)] = jnp.ones((L,), jnp.float32)
    zds = []
    for k in range(NRC):
        rb = s * RPT + k * RCH
        zds.append(pltpu.async_copy(fin, acc_sh.at[pl.ds(rb, RCH)], si0))
        zds.append(pltpu.async_copy(rs_v, rsum_sh.at[pl.ds(rb, RCH)], st0))
        zds.append(pltpu.async_copy(rs_v, deg_sh.at[pl.ds(rb, RCH)], st1))
    for d in zds:
        d.wait()
    plsc.subcore_barrier()

    def issue_idx(g, j):
        base = s * EPT + g * C
        pltpu.async_copy(src_h.at[pl.ds(base, C)], isrc4.at[j], semi[j])
        pltpu.async_copy(dst_h.at[pl.ds(base, C)], idst4.at[j], semi[j])
        pltpu.async_copy(et_h.at[pl.ds(base, C)], iet4.at[j], semi[j])

    def wait_idx(j):
        pltpu.make_async_copy(src_h.at[pl.ds(0, C)], isrc4.at[j],
                              semi[j]).wait()
        pltpu.make_async_copy(dst_h.at[pl.ds(0, C)], idst4.at[j],
                              semi[j]).wait()
        pltpu.make_async_copy(et_h.at[pl.ds(0, C)], iet4.at[j],
                              semi[j]).wait()

    def comp_idx(j):
        for grp in range(C // L):
            sg = isrc4[j, pl.ds(grp * L, L)]
            etg = iet4[j, pl.ds(grp * L, L)]
            dg = idst4[j, pl.ds(grp * L, L)]
            iflat4[j, pl.ds(grp * L, L)] = (sg + c * NP) * R + etg
            ixw4[j, pl.ds(grp * L, L)] = dg + c * NP

    def issue_gathers(j, p):
        pltpu.async_copy(t_h.at[iflat4.at[j]], tval2.at[p], semt[p])
        pltpu.async_copy(s3_h.at[ixw4.at[j]], s3val2.at[p], semt[p])
        pltpu.async_copy(xw_h.at[ixw4.at[j]], xr[p], semg[p])

    def wait_gathers(p):
        pltpu.make_async_copy(t_h.at[pl.ds(0, C)], tval2.at[p],
                              semt[p]).wait()
        pltpu.make_async_copy(s3_h.at[pl.ds(0, C)], s3val2.at[p],
                              semt[p]).wait()
        pltpu.make_async_copy(xw_h.at[pl.ds(0, C)], xr[p], semg[p]).wait()

    def compute_e(p):
        for grp in range(C // L):
            tv = tval2[p, pl.ds(grp * L, L)]
            s3d = s3val2[p, pl.ds(grp * L, L)]
            logit = tv + s3d
            lr = jnp.where(logit >= 0.0, logit, ALPHA * logit)
            ev[p][pl.ds(grp * L, L)] = jnp.exp(-lr)

    def scale(p):
        xrp = xr[p]
        evp = ev[p]

        @plsc.parallel_loop(0, C, step=1)
        def scale_row(i):
            ei = evp[pl.ds(i, L)][0]
            for j in range(NH // L):
                xrp[i, pl.ds(j * L, L)] = xrp[i, pl.ds(j * L, L)] * ei

    def issue_scatters(j, p):
        pltpu.async_copy(xr[p], acc_sh.at[isrc4.at[j]], sems[p], add=True)
        pltpu.async_copy(ev[p].at[pl.ds(0, C)], rsum_sh.at[isrc4.at[j]],
                         sems[p], add=True)
        pltpu.async_copy(ones_v, deg_sh.at[isrc4.at[j]], sems[p], add=True)

    def wait_scatters(p):
        pltpu.make_async_copy(xw_h.at[pl.ds(0, C)], xr[p], sems[p]).wait()
        pltpu.make_async_copy(t_h.at[pl.ds(0, C)], ev[p].at[pl.ds(0, C)],
                              sems[p]).wait()
        pltpu.make_async_copy(t_h.at[pl.ds(0, C)], ones_v, sems[p]).wait()

    def chunk_step(g, r, have_g1, have_g3, have_gm1):
        j, j1, j3 = r % 4, (r + 1) % 4, (r + 3) % 4
        p, p1 = r % 2, (r + 1) % 2
        if have_g1:
            wait_idx(j1)
            comp_idx(j1)
        if have_gm1:
            wait_scatters(p1)
        if have_g3:
            issue_idx(g + 3, j3)
        if have_g1:
            issue_gathers(j1, p1)
        wait_gathers(p)
        compute_e(p)
        scale(p)
        issue_scatters(j, p)

    issue_idx(0, 0)
    issue_idx(1, 1)
    issue_idx(2, 2)
    wait_idx(0)
    comp_idx(0)
    issue_gathers(0, 0)
    for r in range(4):
        chunk_step(r, r, True, True, r >= 1)

    def quad(q, carry):
        for r in range(4):
            chunk_step(q * 4 + r, r, True, True, True)
        return carry
    lax.fori_loop(1, NCHUNK // 4 - 1, quad, 0)

    for r in range(4):
        chunk_step(NCHUNK - 4 + r, r, r < 3, r < 1, True)
    wait_scatters(1)
    plsc.subcore_barrier()

    for k in range(NRC):
        rb = s * RPT + k * RCH
        d1 = pltpu.async_copy(acc_sh.at[pl.ds(rb, RCH)], fin, si0)
        d2 = pltpu.async_copy(rsum_sh.at[pl.ds(rb, RCH)], rs_v, st0)
        d3 = pltpu.async_copy(deg_sh.at[pl.ds(rb, RCH)], deg_v, st1)
        d1.wait()
        d2.wait()
        d3.wait()
        for gg in range(RCH // L):
            rs = rs_v[pl.ds(gg * L, L)]
            rs = jnp.where(rs == 0.0, 1e-12, rs)
            degv = deg_v[pl.ds(gg * L, L)] + 1.0
            ii = plsc.bitcast(degv, jnp.int32)
            ii = jnp.int32(0x5F3759DF) - lax.shift_right_arithmetic(ii, 1)
            y = plsc.bitcast(ii, jnp.float32)
            for _ in range(3):
                y = y * (1.5 - 0.5 * degv * y * y)
            invc[pl.ds(gg * L, L)] = 1.0 / rs
            disc[pl.ds(gg * L, L)] = y

        @plsc.parallel_loop(0, RCH, step=1)
        def fin_row(i):
            iv = invc[pl.ds(i, L)][0]
            for j in range(NH // L):
                v = fin[i, pl.ds(j * L, L)] * iv
                fin[i, pl.ds(j * L, L)] = jnp.where(v > 0.0, v,
                                                    jnp.exp(v) - 1.0)
        pltpu.sync_copy(fin, hl_h.at[pl.ds(c * NP + rb, RCH)])

        @plsc.parallel_loop(0, RCH, step=1)
        def scl_row(i):
            di = disc[pl.ds(i, L)][0]
            for j in range(NH // L):
                fin[i, pl.ds(j * L, L)] = fin[i, pl.ds(j * L, L)] * di
        pltpu.sync_copy(fin, hls_h.at[pl.ds(c * NP + rb, RCH)])

        @pl.when(c == 0)
        def _():
            pltpu.sync_copy(disc.at[pl.ds(0, RCH)], dis_h.at[pl.ds(rb, RCH)])


_k1 = functools.partial(
    pl.kernel,
    out_type=(jax.ShapeDtypeStruct((HEADS * NP, NH), jnp.float32),
              jax.ShapeDtypeStruct((HEADS * NP, NH), jnp.float32),
              jax.ShapeDtypeStruct((NP,), jnp.float32)),
    mesh=_MESH,
    compiler_params=pltpu.CompilerParams(needs_layout_passes=False),
    scratch_types=[
        pltpu.VMEM_SHARED((NP, NH), jnp.float32),
        pltpu.VMEM_SHARED((NP,), jnp.float32),
        pltpu.VMEM_SHARED((NP,), jnp.float32),
        pltpu.VMEM((4, C), jnp.int32),
        pltpu.VMEM((4, C), jnp.int32),
        pltpu.VMEM((4, C), jnp.int32),
        pltpu.VMEM((4, C), jnp.int32),
        pltpu.VMEM((4, C), jnp.int32),
        pltpu.VMEM((2, C), jnp.float32),
        pltpu.VMEM((2, C), jnp.float32),
        pltpu.VMEM((C, NH), jnp.float32),
        pltpu.VMEM((C, NH), jnp.float32),
        pltpu.VMEM((C + L,), jnp.float32),
        pltpu.VMEM((C + L,), jnp.float32),
        pltpu.VMEM((C,), jnp.float32),
        pltpu.VMEM((RCH, NH), jnp.float32),
        pltpu.VMEM((RCH + L,), jnp.float32),
        pltpu.VMEM((RCH + L,), jnp.float32),
        pltpu.VMEM((RCH,), jnp.float32),
        pltpu.VMEM((RCH,), jnp.float32),
        pltpu.SemaphoreType.DMA,
        pltpu.SemaphoreType.DMA,
        pltpu.SemaphoreType.DMA,
        pltpu.SemaphoreType.DMA,
        pltpu.SemaphoreType.DMA,
        pltpu.SemaphoreType.DMA,
        pltpu.SemaphoreType.DMA,
        pltpu.SemaphoreType.DMA,
        pltpu.SemaphoreType.DMA,
        pltpu.SemaphoreType.DMA,
    ],
)(_k1_body)


def _k2_body(src_h, dst_h, dis_h, hls_h,
             hg_h, a_h, b_h,
             acc_sh, disf, isrc4, idst4, ixw4, xra, xrb, fin, prev,
             si0, si1, si2, si3, sg0, sg1, ss0, ss1):
    c = lax.axis_index("c")
    s = lax.axis_index("s")
    semi = [si0, si1, si2, si3]
    semg = [sg0, sg1]
    sems = [ss0, ss1]
    xr = [xra, xrb]

    def issue_idx(g, j):
        base = s * EPT + g * C2
        pltpu.async_copy(src_h.at[pl.ds(base, C2)], isrc4.at[j], semi[j])
        pltpu.async_copy(dst_h.at[pl.ds(base, C2)], idst4.at[j], semi[j])

    def wait_idx(j):
        pltpu.make_async_copy(src_h.at[pl.ds(0, C2)], isrc4.at[j],
                              semi[j]).wait()
        pltpu.make_async_copy(dst_h.at[pl.ds(0, C2)], idst4.at[j],
                              semi[j]).wait()

    def comp_ixw(j):
        for grp in range(C2 // L):
            dg = idst4[j, pl.ds(grp * L, L)]
            ixw4[j, pl.ds(grp * L, L)] = dg + c * NP

    hops = [(hls_h, a_h), (a_h, b_h), (b_h, hg_h)]
    for hop, (src_ref, dst_ref) in enumerate(hops):
        _zero_buf(fin, RCH, NH)
        zds = []
        for k in range(NRC):
            zds.append(pltpu.async_copy(
                fin, acc_sh.at[pl.ds(s * RPT + k * RCH, RCH)], si0))
        for d in zds:
            d.wait()
        plsc.subcore_barrier()

        def issue_gather(j, p):
            pltpu.async_copy(src_ref.at[ixw4.at[j]], xr[p], semg[p])

        def wait_gather(p):
            pltpu.make_async_copy(src_ref.at[pl.ds(0, C2)], xr[p],
                                  semg[p]).wait()

        def issue_scatter(j, p):
            pltpu.async_copy(xr[p], acc_sh.at[isrc4.at[j]], sems[p], add=True)

        def wait_scatter(p):
            pltpu.make_async_copy(src_ref.at[pl.ds(0, C2)], xr[p],
                                  sems[p]).wait()

        def chunk_step(g, r, have_g1, have_g3, have_gm1):
            j, j1, j3 = r % 4, (r + 1) % 4, (r + 3) % 4
            p, p1 = r % 2, (r + 1) % 2
            if have_g1:
                wait_idx(j1)
                comp_ixw(j1)
            if have_gm1:
                wait_scatter(p1)
            if have_g3:
                issue_idx(g + 3, j3)
            if have_g1:
                issue_gather(j1, p1)
            wait_gather(p)
            issue_scatter(j, p)

        issue_idx(0, 0)
        issue_idx(1, 1)
        issue_idx(2, 2)
        wait_idx(0)
        comp_ixw(0)
        issue_gather(0, 0)
        for r in range(4):
            chunk_step(r, r, True, True, r >= 1)

        def quad(q, carry):
            for r in range(4):
                chunk_step(q * 4 + r, r, True, True, True)
            return carry
        lax.fori_loop(1, NCH2 // 4 - 1, quad, 0)

        for r in range(4):
            g = NCH2 - 4 + r
            chunk_step(g, r, r < 3, r < 1, True)
        wait_scatter(1)
        plsc.subcore_barrier()

        for k in range(NRC):
            rb = s * RPT + k * RCH
            d1 = pltpu.async_copy(acc_sh.at[pl.ds(rb, RCH)], fin, si0)
            d2 = pltpu.async_copy(src_ref.at[pl.ds(c * NP + rb, RCH)], prev,
                                  si1)
            d3 = pltpu.async_copy(dis_h.at[pl.ds(rb, RCH)],
                                  disf.at[pl.ds(0, RCH)], si2)
            d1.wait()
            d2.wait()
            d3.wait()

            @plsc.parallel_loop(0, RCH, step=1)
            def fin_row(i):
                di = disf[pl.ds(i, L)][0]
                sw = di * di if hop < K - 1 else di
                for j in range(NH // L):
                    fin[i, pl.ds(j * L, L)] = (fin[i, pl.ds(j * L, L)] +
                                               prev[i, pl.ds(j * L, L)]) * sw
            pltpu.sync_copy(fin, dst_ref.at[pl.ds(c * NP + rb, RCH)])
        plsc.subcore_barrier()


_k2 = functools.partial(
    pl.kernel,
    out_type=(jax.ShapeDtypeStruct((HEADS * NP, NH), jnp.float32),
              jax.ShapeDtypeStruct((HEADS * NP, NH), jnp.float32),
              jax.ShapeDtypeStruct((HEADS * NP, NH), jnp.float32)),
    mesh=_MESH,
    compiler_params=pltpu.CompilerParams(needs_layout_passes=False),
    scratch_types=[
        pltpu.VMEM_SHARED((NP, NH), jnp.float32),
        pltpu.VMEM((RCH + L,), jnp.float32),
        pltpu.VMEM((4, C2), jnp.int32),
        pltpu.VMEM((4, C2), jnp.int32),
        pltpu.VMEM((4, C2), jnp.int32),
        pltpu.VMEM((C2, NH), jnp.float32),
        pltpu.VMEM((C2, NH), jnp.float32),
        pltpu.VMEM((RCH, NH), jnp.float32),
        pltpu.VMEM((RCH, NH), jnp.float32),
        pltpu.SemaphoreType.DMA,
        pltpu.SemaphoreType.DMA,
        pltpu.SemaphoreType.DMA,
        pltpu.SemaphoreType.DMA,
        pltpu.SemaphoreType.DMA,
        pltpu.SemaphoreType.DMA,
        pltpu.SemaphoreType.DMA,
        pltpu.SemaphoreType.DMA,
    ],
)(_k2_body)


def _fuse_body(hl_ref, hg_ref, w_ref, b_ref, out_ref):
    acc = b_ref[...]
    acc = acc + _dot(hl_ref[0], w_ref[0:NH, :], ((1,), (0,)))
    acc = acc + _dot(hl_ref[1], w_ref[NH:2 * NH, :], ((1,), (0,)))
    acc = acc + _dot(hg_ref[0], w_ref[2 * NH:3 * NH, :], ((1,), (0,)))
    acc = acc + _dot(hg_ref[1], w_ref[3 * NH:4 * NH, :], ((1,), (0,)))
    out_ref[...] = jnp.where(acc > 0.0, acc, jnp.exp(acc) - 1.0)


def _fuse(hl, hg, fw, fb):
    grid = NP // _BR
    return pl.pallas_call(
        _fuse_body,
        grid=(grid,),
        in_specs=[
            pl.BlockSpec((HEADS, _BR, NH), lambda i: (0, i, 0)),
            pl.BlockSpec((HEADS, _BR, NH), lambda i: (0, i, 0)),
            pl.BlockSpec((2 * HEADS * NH, HEADS * NH), lambda i: (0, 0)),
            pl.BlockSpec((1, HEADS * NH), lambda i: (0, 0)),
        ],
        out_specs=pl.BlockSpec((_BR, HEADS * NH), lambda i: (i, 0)),
        out_shape=jax.ShapeDtypeStruct((NP, HEADS * NH), jnp.float32),
    )(hl, hg, fw, fb)


@jax.jit
def kernel(entity_embeddings, relation_embeddings, edge_list, edge_type,
           thW1, thb1, thW2, thb2, c_r, W_att, W_r, a_att, W_out,
           fuse_W, fuse_b):
    pad = N + (jnp.arange(EPAD - E, dtype=jnp.int32) % (NP - N))
    src = jnp.concatenate([edge_list[0], pad])
    dst = jnp.concatenate([edge_list[1], pad])
    et = jnp.concatenate([edge_type, jnp.zeros((EPAD - E,), jnp.int32)])
    x_pad = jnp.pad(entity_embeddings, ((0, NP - N), (0, 0)))
    t_tab, s3_tab, xw_tab, x_rel = _prep(
        x_pad, relation_embeddings,
        thW1, thb1.reshape(HEADS, 1, 512), thW2, thb2.reshape(HEADS, 1, NF),
        c_r, W_att, W_r, a_att, W_out)
    s3_tab = s3_tab.reshape(HEADS * NP)
    t_tab = t_tab.reshape(HEADS * NP * R)
    xw_tab = xw_tab.reshape(HEADS * NP, NH)
    hl, hls, dis = _k1(src, dst, et, t_tab, s3_tab, xw_tab)
    hg, _, _ = _k2(src, dst, dis, hls)
    h_fused = _fuse(hl.reshape(HEADS, NP, NH), hg.reshape(HEADS, NP, NH),
                    fuse_W, fuse_b.reshape(1, HEADS * NH))
    return (h_fused[:N], x_rel)
